# baseline (speedup 1.0000x reference)
import functools

import jax
import jax.numpy as jnp
from jax import lax
from jax.experimental import pallas as pl
from jax.experimental.pallas import tpu as pltpu
from jax.experimental.pallas import tpu_sc as plsc

N = 50000
N1 = 25000
N2 = 6250
E1 = 400000
E2 = 200000
B = 16

NC = 2
NS = 16
NW = NC * NS

N1P = 25088
N2P = 6400
E2P = 200704
PAD_DST = 1 << 20

_NEG_INF = float("-inf")


def _mesh():
    return plsc.VectorSubcoreMesh(core_axis_name="c", subcore_axis_name="s")


def _wid():
    return lax.axis_index("s") * NC + lax.axis_index("c")


def _tile8(b):
    return jnp.broadcast_to(b[None, :], (8, b.shape[0]))


def _gather_h1(T, TP, idx1, src1, dst1):
    CHUNKS = E1 // 128
    KMAX = (CHUNKS + NW - 1) // NW

    @functools.partial(
        pl.kernel,
        out_type=jax.ShapeDtypeStruct((E1, 16), jnp.float32),
        mesh=_mesh(),
        compiler_params=pltpu.CompilerParams(use_tc_tiling_on_sc=False, needs_layout_passes=False),
        scratch_types=[
            pltpu.VMEM((128,), jnp.int32),
            pltpu.VMEM((128,), jnp.int32),
            pltpu.VMEM((128,), jnp.int32),
            pltpu.VMEM((128, 16), jnp.float32),
            pltpu.VMEM((128, 16), jnp.float32),
            pltpu.VMEM((128, 16), jnp.float32),
            pltpu.SemaphoreType.DMA,
            pltpu.SemaphoreType.DMA,
        ],
    )
    def k(T_hbm, TP_hbm, idx1_hbm, src1_hbm, dst1_hbm, h1_hbm,
          sv, dv, ds_v, ra, rb, hb, semA, semB):
        w = _wid()

        def chunk_body(kk, _):
            chunk = w + kk * NW

            @pl.when(chunk < CHUNKS)
            def _():
                base = chunk * 128
                pltpu.sync_copy(src1_hbm.at[pl.ds(base, 128)], sv)
                pltpu.sync_copy(dst1_hbm.at[pl.ds(base, 128)], dv)
                pltpu.async_copy(idx1_hbm.at[dv], ds_v, semA).wait()
                cpA = pltpu.async_copy(T_hbm.at[sv], ra, semA)
                cpB = pltpu.async_copy(TP_hbm.at[ds_v], rb, semB)
                cpA.wait()
                cpB.wait()

                def sub_body(i, c):
                    hb[i, :] = ra[i, :] - rb[i, :]
                    return c
                lax.fori_loop(0, 128, sub_body, 0)
                pltpu.sync_copy(hb, h1_hbm.at[pl.ds(base, 128)])
            return 0

        lax.fori_loop(0, KMAX, chunk_body, 0)

    return k(T, TP, idx1, src1, dst1)


def _scatter_max(Y, dst, nout_p, F, NCHUNK):
    RPW = nout_p // NW
    GB = 64
    CH = 1024
    assert NCHUNK % 2 == 0

    @functools.partial(
        pl.kernel,
        out_type=jax.ShapeDtypeStruct((nout_p, F), jnp.float32),
        mesh=_mesh(),
        compiler_params=pltpu.CompilerParams(use_tc_tiling_on_sc=False, needs_layout_passes=False),
        scratch_types=[
            pltpu.VMEM((RPW + 1, F), jnp.float32),
            pltpu.VMEM((CH,), jnp.int32),
            pltpu.VMEM((CH,), jnp.int32),
            pltpu.VMEM((CH + GB,), jnp.int32),
            pltpu.VMEM((CH + GB,), jnp.int32),
            pltpu.VMEM((CH + GB,), jnp.int32),
            pltpu.VMEM((CH + GB,), jnp.int32),
            pltpu.VMEM((GB, F), jnp.float32),
            pltpu.VMEM((GB, F), jnp.float32),
            pltpu.VMEM((GB,), jnp.int32),
            pltpu.VMEM((GB,), jnp.int32),
            pltpu.SemaphoreType.DMA,
            pltpu.SemaphoreType.DMA,
        ],
    )
    def k(y_hbm, dst_hbm, out_hbm, acc, dstb0, dstb1, midb0, midb1,
          lrb0, lrb1, rows0, rows1, mh0, mh1, gsem0, gsem1):
        w = _wid()
        lo = w * RPW

        def init_body(r, c):
            for cc in range(F // 16):
                acc[r, pl.ds(cc * 16, 16)] = jnp.full((16,), _NEG_INF, jnp.float32)
            return c
        lax.fori_loop(0, RPW + 1, init_body, 0)

        def scan_issue(kk, dstb, midb, lrb, rowsb, mh, gsem):
            pltpu.sync_copy(dst_hbm.at[pl.ds(kk * CH, CH)], dstb)

            def scan_body(j, ptr):
                groups = []
                for g in range(8):
                    off = j * 128 + g * 16
                    d = dstb[pl.ds(off, 16)]
                    lrel = d - lo
                    m = (lrel >= 0) & (lrel < RPW)
                    eid = kk * CH + off + lax.iota(jnp.int32, 16)
                    key = jnp.where(m, lrel, jnp.int32(1 << 30))
                    sk, sval = plsc.sort_key_val(key, eid)
                    cnt = plsc.all_reduce_population_count(m)
                    groups.append((sk, sval, cnt))
                for sk, sval, cnt in groups:
                    lrb[pl.ds(ptr, 16)] = sk
                    midb[pl.ds(ptr, 16)] = sval
                    ptr = ptr + cnt[0]
                return ptr

            ptr = lax.fori_loop(0, CH // 128, scan_body, jnp.int32(0))

            zz = jnp.zeros((16,), jnp.int32)
            tt = jnp.full((16,), RPW, jnp.int32)
            for q in range(GB // 16):
                midb[pl.ds(ptr + q * 16, 16)] = zz
                lrb[pl.ds(ptr + q * 16, 16)] = tt

            for q in range(GB // 16):
                mh[pl.ds(q * 16, 16)] = midb[pl.ds(q * 16, 16)]
            pltpu.async_copy(y_hbm.at[mh], rowsb, gsem)
            return jnp.maximum(lax.div(ptr + (GB - 1), jnp.int32(GB)),
                               jnp.int32(1))

        def acc_batch(bb, lrb, rref):
            def jj_body(jj, c):
                lv = lrb[pl.ds(bb * GB + jj * 16, 16)]
                for i in range(16):
                    lr = lv[i]
                    for cc in range(F // 16):
                        sl = pl.ds(cc * 16, 16)
                        acc[lr, sl] = jnp.maximum(acc[lr, sl],
                                                  rref[jj * 16 + i, sl])
                return c
            lax.fori_loop(0, GB // 16, jj_body, 0)

        def drain(nsub, midb, lrb, rowsb, mh, gsem):
            pltpu.make_async_copy(y_hbm.at[mh], rowsb, gsem).wait()
            acc_batch(0, lrb, rowsb)

            def tail_body(bb, c):
                for q in range(GB // 16):
                    mh[pl.ds(q * 16, 16)] = midb[pl.ds(bb * GB + q * 16, 16)]
                pltpu.async_copy(y_hbm.at[mh], rowsb, gsem).wait()
                acc_batch(bb, lrb, rowsb)
                return c
            lax.fori_loop(1, nsub, tail_body, 0)

        ns_a = scan_issue(0, dstb0, midb0, lrb0, rows0, mh0, gsem0)

        def pair_body(kk2, ns_a):
            ns_b = scan_issue(2 * kk2 + 1, dstb1, midb1, lrb1, rows1, mh1, gsem1)
            drain(ns_a, midb0, lrb0, rows0, mh0, gsem0)
            ns_a2 = scan_issue(2 * kk2 + 2, dstb0, midb0, lrb0, rows0, mh0, gsem0)
            drain(ns_b, midb1, lrb1, rows1, mh1, gsem1)
            return ns_a2

        ns_a = lax.fori_loop(0, NCHUNK // 2 - 1, pair_body, ns_a)
        ns_b = scan_issue(NCHUNK - 1, dstb1, midb1, lrb1, rows1, mh1, gsem1)
        drain(ns_a, midb0, lrb0, rows0, mh0, gsem0)
        drain(ns_b, midb1, lrb1, rows1, mh1, gsem1)

        def fin_body(r, c):
            for cc in range(F // 16):
                sl = pl.ds(cc * 16, 16)
                v = acc[r, sl]
                acc[r, sl] = jnp.where(v > _NEG_INF, v, jnp.float32(0.0))
            return c
        lax.fori_loop(0, RPW, fin_body, 0)
        pltpu.sync_copy(acc.at[pl.ds(0, RPW)], out_hbm.at[pl.ds(lo, RPW)])

    return k(Y, dst)


def _gather_l2(X1, TP, idx1, idx2p, batch, src2p, dst2p):
    CHUNKS = E2P // 128
    KMAX = CHUNKS // NW
    UCH = N2P // 128

    out_type = [
        jax.ShapeDtypeStruct((E2P, 128), jnp.float32),
        jax.ShapeDtypeStruct((E2P, 16), jnp.float32),
        jax.ShapeDtypeStruct((N2P, 16), jnp.float32),
        jax.ShapeDtypeStruct((N2P,), jnp.int32),
    ]

    @functools.partial(
        pl.kernel,
        out_type=out_type,
        mesh=_mesh(),
        compiler_params=pltpu.CompilerParams(use_tc_tiling_on_sc=False, needs_layout_passes=False),
        scratch_types=[
            pltpu.VMEM((128,), jnp.int32),
            pltpu.VMEM((128,), jnp.int32),
            pltpu.VMEM((128,), jnp.int32),
            pltpu.VMEM((128,), jnp.int32),
            pltpu.VMEM((128,), jnp.int32),
            pltpu.VMEM((128,), jnp.int32),
            pltpu.VMEM((128, 128), jnp.float32),
            pltpu.VMEM((128, 16), jnp.float32),
            pltpu.VMEM((128, 16), jnp.float32),
            pltpu.VMEM((128, 16), jnp.float32),
            pltpu.VMEM((128,), jnp.int32),
            pltpu.SemaphoreType.DMA,
            pltpu.SemaphoreType.DMA,
            pltpu.SemaphoreType.DMA,
        ],
    )
    def k(x1_hbm, tp_hbm, idx1_hbm, idx2_hbm, batch_hbm, src2_hbm, dst2_hbm,
          h2x_hbm, h2d_hbm, u2_hbm, b2_hbm,
          sv, dv, dc, t1, csv, cdv, rx, rp1, rp2, hd, bv,
          semX, semA, semB):
        w = _wid()

        def chunk_body(kk, _):
            base = (w + kk * NW) * 128
            pltpu.sync_copy(src2_hbm.at[pl.ds(base, 128)], sv)
            pltpu.sync_copy(dst2_hbm.at[pl.ds(base, 128)], dv)
            cpX = pltpu.async_copy(x1_hbm.at[sv], rx, semX)
            for j in range(8):
                s16 = pl.ds(j * 16, 16)
                dc[s16] = jnp.minimum(dv[s16], jnp.int32(N2 - 1))
            cpS = pltpu.async_copy(idx1_hbm.at[sv], csv, semA)
            pltpu.async_copy(idx2_hbm.at[dc], t1, semB).wait()
            pltpu.async_copy(idx1_hbm.at[t1], cdv, semB).wait()
            cpS.wait()
            cpA = pltpu.async_copy(tp_hbm.at[csv], rp1, semA)
            cpB = pltpu.async_copy(tp_hbm.at[cdv], rp2, semB)
            cpX.wait()
            pltpu.sync_copy(rx, h2x_hbm.at[pl.ds(base, 128)])
            cpA.wait()
            cpB.wait()

            def sub_body(i, c):
                hd[i, :] = rp1[i, :] - rp2[i, :]
                return c
            lax.fori_loop(0, 128, sub_body, 0)
            pltpu.sync_copy(hd, h2d_hbm.at[pl.ds(base, 128)])
            return 0

        lax.fori_loop(0, KMAX, chunk_body, 0)

        def u_body(kk, _):
            chunk = w + kk * NW

            @pl.when(chunk < UCH)
            def _():
                base = chunk * 128
                pltpu.sync_copy(idx2_hbm.at[pl.ds(base, 128)], t1)
                pltpu.async_copy(idx1_hbm.at[t1], cdv, semA).wait()
                pltpu.async_copy(batch_hbm.at[cdv], dc, semA).wait()
                pltpu.async_copy(tp_hbm.at[cdv], rp1, semB).wait()
                for j in range(8):
                    s16 = pl.ds(j * 16, 16)
                    rowid = base + j * 16 + lax.iota(jnp.int32, 16)
                    bv[s16] = jnp.where(rowid < N2, dc[s16], jnp.int32(B))
                pltpu.sync_copy(rp1, u2_hbm.at[pl.ds(base, 128)])
                pltpu.sync_copy(bv, b2_hbm.at[pl.ds(base, 128)])
            return 0

        lax.fori_loop(0, 2, u_body, 0)

    return k(X1, TP, idx1, idx2p, batch, src2p, dst2p)


def _mlp1_tc(H1, W1, b1, W2, b2, W3, b3):
    BE = 2000
    G = E1 // BE

    def body(h_ref, w1, bb1, w2, bb2, w3, bb3, out_ref):
        h = h_ref[...]
        h = jnp.maximum(jnp.dot(h, w1[...], preferred_element_type=jnp.float32)
                        + bb1[0:1, :], 0.0)
        h = jnp.maximum(jnp.dot(h, w2[...], preferred_element_type=jnp.float32)
                        + bb2[0:1, :], 0.0)
        out_ref[...] = (jnp.dot(h, w3[...], preferred_element_type=jnp.float32)
                        + bb3[0:1, :])

    return pl.pallas_call(
        body,
        grid=(G,),
        in_specs=[
            pl.BlockSpec((BE, 16), lambda i: (i, 0)),
            pl.BlockSpec((16, 64), lambda i: (0, 0)),
            pl.BlockSpec((8, 64), lambda i: (0, 0)),
            pl.BlockSpec((64, 64), lambda i: (0, 0)),
            pl.BlockSpec((8, 64), lambda i: (0, 0)),
            pl.BlockSpec((64, 128), lambda i: (0, 0)),
            pl.BlockSpec((8, 128), lambda i: (0, 0)),
        ],
        out_specs=pl.BlockSpec((BE, 128), lambda i: (i, 0)),
        out_shape=jax.ShapeDtypeStruct((E1, 128), jnp.float32),
    )(H1, W1, b1, W2, b2, W3, b3)


def _mlp2_tc(H2x, H2d, Wa, Wb, b1, W2, b2, W3, b3):
    BE = 4096
    G = E2P // BE

    def body(hx_ref, hd_ref, wa, wb, bb1, w2, bb2, w3, bb3, out_ref):
        h = jnp.dot(hx_ref[...], wa[...], preferred_element_type=jnp.float32)
        h = h + jnp.dot(hd_ref[...], wb[...], preferred_element_type=jnp.float32)
        h = jnp.maximum(h + bb1[0:1, :], 0.0)
        h = jnp.maximum(jnp.dot(h, w2[...], preferred_element_type=jnp.float32)
                        + bb2[0:1, :], 0.0)
        out_ref[...] = (jnp.dot(h, w3[...], preferred_element_type=jnp.float32)
                        + bb3[0:1, :])

    return pl.pallas_call(
        body,
        grid=(G,),
        in_specs=[
            pl.BlockSpec((BE, 128), lambda i: (i, 0)),
            pl.BlockSpec((BE, 16), lambda i: (i, 0)),
            pl.BlockSpec((128, 128), lambda i: (0, 0)),
            pl.BlockSpec((16, 128), lambda i: (0, 0)),
            pl.BlockSpec((8, 128), lambda i: (0, 0)),
            pl.BlockSpec((128, 128), lambda i: (0, 0)),
            pl.BlockSpec((8, 128), lambda i: (0, 0)),
            pl.BlockSpec((128, 256), lambda i: (0, 0)),
            pl.BlockSpec((8, 256), lambda i: (0, 0)),
        ],
        out_specs=pl.BlockSpec((BE, 256), lambda i: (i, 0)),
        out_shape=jax.ShapeDtypeStruct((E2P, 256), jnp.float32),
    )(H2x, H2d, Wa, Wb, b1, W2, b2, W3, b3)


def _final_tc(X2, U2, B2r, Wa, Wb, b1, W2, b2, W3, b3,
              Wh1, bh1, Wh2, bh2, Wh3, bh3, Wp1, bp1, Wp2, bp2):
    BR = 800
    G = N2P // BR

    def body(x2_ref, u2_ref, b2_ref, wa, wb, bb1, w2, bb2, w3, bb3,
             wh1, bbh1, wh2, bbh2, wh3, bbh3, wp1, bbp1, wp2, bbp2,
             out_ref, acc):
        step = pl.program_id(0)

        @pl.when(step == 0)
        def _():
            acc[...] = jnp.full((B, 1024), _NEG_INF, jnp.float32)

        h = jnp.dot(x2_ref[...], wa[...], preferred_element_type=jnp.float32)
        h = h + jnp.dot(u2_ref[...], wb[...], preferred_element_type=jnp.float32)
        h = jnp.maximum(h + bb1[0:1, :], 0.0)
        h = jnp.maximum(jnp.dot(h, w2[...], preferred_element_type=jnp.float32)
                        + bb2[0:1, :], 0.0)
        h = jnp.dot(h, w3[...], preferred_element_type=jnp.float32) + bb3[0:1, :]
        bid = b2_ref[...][:, 0:1]
        parts = []
        for bb in range(B):
            hm = jnp.where(bid == bb, h, _NEG_INF)
            parts.append(jnp.max(hm, axis=0, keepdims=True))
        acc[...] = jnp.maximum(acc[...], jnp.concatenate(parts, axis=0))

        @pl.when(step == G - 1)
        def _():
            xg = acc[...]
            xg = jnp.where(xg > _NEG_INF, xg, 0.0)
            f = jnp.maximum(jnp.dot(xg, wh1[...], preferred_element_type=jnp.float32)
                            + bbh1[0:1, :], 0.0)
            f = jnp.maximum(jnp.dot(f, wh2[...], preferred_element_type=jnp.float32)
                            + bbh2[0:1, :], 0.0)
            f = jnp.dot(f, wh3[...], preferred_element_type=jnp.float32) + bbh3[0:1, :]
            f = jnp.maximum(jnp.dot(f, wp1[...], preferred_element_type=jnp.float32)
                            + bbp1[0:1, :], 0.0)
            out_ref[...] = (jnp.dot(f, wp2[...], preferred_element_type=jnp.float32)
                            + bbp2[0:1, :])

    full = lambda s: pl.BlockSpec(s, lambda i: (0, 0))
    return pl.pallas_call(
        body,
        grid=(G,),
        in_specs=[
            pl.BlockSpec((BR, 256), lambda i: (i, 0)),
            pl.BlockSpec((BR, 16), lambda i: (i, 0)),
            pl.BlockSpec((BR, 128), lambda i: (i, 0)),
            full((256, 256)), full((16, 256)), full((8, 256)),
            full((256, 512)), full((8, 512)),
            full((512, 1024)), full((8, 1024)),
            full((1024, 512)), full((8, 512)),
            full((512, 256)), full((8, 256)),
            full((256, 32)), full((8, 32)),
            full((32, 32)), full((8, 32)),
            full((32, 8)), full((8, 8)),
        ],
        out_specs=pl.BlockSpec((B, 8), lambda i: (0, 0)),
        out_shape=jax.ShapeDtypeStruct((B, 8), jnp.float32),
        scratch_shapes=[pltpu.VMEM((B, 1024), jnp.float32)],
    )(X2, U2, B2r, Wa, Wb, b1, W2, b2, W3, b3,
      Wh1, bh1, Wh2, bh2, Wh3, bh3, Wp1, bp1, Wp2, bp2)


def kernel(x, pos, batch, idx1, src1, dst1, idx2, src2, dst2,
           params1, params2, params3, params_head, params_policy):
    f32 = jnp.float32
    (W11, b11), (W12, b12), (W13, b13) = params1
    (W21, b21), (W22, b22), (W23, b23) = params2
    (W31, b31), (W32, b32), (W33, b33) = params3
    (Wh1, bh1), (Wh2, bh2), (Wh3, bh3) = params_head
    (Wp1, bp1), (Wp2, bp2) = params_policy

    T = jnp.concatenate([x, pos, jnp.zeros((N, 1), f32)], axis=1)
    TP = jnp.concatenate([jnp.zeros((N, 12), f32), pos,
                          jnp.zeros((N, 1), f32)], axis=1)

    H1 = _gather_h1(T, TP, idx1, src1, dst1)
    W1p = jnp.concatenate([W11, jnp.zeros((1, 64), f32)], axis=0)
    Y1 = _mlp1_tc(H1, W1p, _tile8(b11), W12, _tile8(b12), W13, _tile8(b13))
    dst1p = jnp.concatenate([dst1, jnp.full((409600 - E1,), PAD_DST, jnp.int32)])
    X1 = _scatter_max(Y1, dst1p, N1P, 128, 400)

    idx2p = jnp.concatenate([idx2, jnp.zeros((N2P - N2,), jnp.int32)])
    src2p = jnp.concatenate([src2, jnp.zeros((E2P - E2,), jnp.int32)])
    dst2p = jnp.concatenate([dst2, jnp.full((E2P - E2,), PAD_DST, jnp.int32)])
    H2x, H2d, U2, B2 = _gather_l2(X1, TP, idx1, idx2p, batch, src2p, dst2p)

    W2a = W21[:128, :]
    W2b = jnp.zeros((16, 128), f32).at[12:15, :].set(W21[128:131, :])
    Y2 = _mlp2_tc(H2x, H2d, W2a, W2b, _tile8(b21), W22, _tile8(b22),
                  W23, _tile8(b23))
    X2 = _scatter_max(Y2, dst2p, N2P, 256, 196)

    W3a = W31[:256, :]
    W3b = jnp.zeros((16, 256), f32).at[12:15, :].set(W31[256:259, :])
    B2r = jnp.broadcast_to(B2[:, None], (N2P, 128))
    logits = _final_tc(X2, U2, B2r, W3a, W3b, _tile8(b31), W32, _tile8(b32),
                       W33, _tile8(b33), Wh1, _tile8(bh1), Wh2, _tile8(bh2),
                       Wh3, _tile8(bh3), Wp1, _tile8(bp1), Wp2, _tile8(bp2))
    return logits


# speedup vs baseline: 1.9410x; 1.9410x over previous
import functools

import jax
import jax.numpy as jnp
from jax import lax
from jax.experimental import pallas as pl
from jax.experimental.pallas import tpu as pltpu
from jax.experimental.pallas import tpu_sc as plsc

N = 50000
N1 = 25000
N2 = 6250
E1 = 400000
E2 = 200000
B = 16

NC = 2
NS = 16
NW = NC * NS

N1P = 25088
N2P = 6400
E2P = 200704
PAD_DST = 1 << 20

_NEG_INF = float("-inf")


def _mesh():
    return plsc.VectorSubcoreMesh(core_axis_name="c", subcore_axis_name="s")


def _wid():
    return lax.axis_index("s") * NC + lax.axis_index("c")


def _tile8(b):
    return jnp.broadcast_to(b[None, :], (8, b.shape[0]))


def _gather_h1(T, TP, idx1, src1, dst1):
    CHUNKS = E1 // 128
    KMAX = (CHUNKS + NW - 1) // NW

    @functools.partial(
        pl.kernel,
        out_type=jax.ShapeDtypeStruct((E1, 16), jnp.float32),
        mesh=_mesh(),
        compiler_params=pltpu.CompilerParams(use_tc_tiling_on_sc=False, needs_layout_passes=False),
        scratch_types=[
            pltpu.VMEM((128,), jnp.int32),
            pltpu.VMEM((128,), jnp.int32),
            pltpu.VMEM((128,), jnp.int32),
            pltpu.VMEM((128, 16), jnp.float32),
            pltpu.VMEM((128, 16), jnp.float32),
            pltpu.VMEM((128, 16), jnp.float32),
            pltpu.SemaphoreType.DMA,
            pltpu.SemaphoreType.DMA,
        ],
    )
    def k(T_hbm, TP_hbm, idx1_hbm, src1_hbm, dst1_hbm, h1_hbm,
          sv, dv, ds_v, ra, rb, hb, semA, semB):
        w = _wid()

        def chunk_body(kk, _):
            chunk = w + kk * NW

            @pl.when(chunk < CHUNKS)
            def _():
                base = chunk * 128
                pltpu.sync_copy(src1_hbm.at[pl.ds(base, 128)], sv)
                pltpu.sync_copy(dst1_hbm.at[pl.ds(base, 128)], dv)
                pltpu.async_copy(idx1_hbm.at[dv], ds_v, semA).wait()
                cpA = pltpu.async_copy(T_hbm.at[sv], ra, semA)
                cpB = pltpu.async_copy(TP_hbm.at[ds_v], rb, semB)
                cpA.wait()
                cpB.wait()

                def sub_body(i, c):
                    hb[i, :] = ra[i, :] - rb[i, :]
                    return c
                lax.fori_loop(0, 128, sub_body, 0)
                pltpu.sync_copy(hb, h1_hbm.at[pl.ds(base, 128)])
            return 0

        lax.fori_loop(0, KMAX, chunk_body, 0)

    return k(T, TP, idx1, src1, dst1)


def _scatter_max(Y, dst, nout_p, F, NCHUNK):
    RPW = nout_p // NW
    GB = 64
    CH = 2048

    @functools.partial(
        pl.kernel,
        out_type=jax.ShapeDtypeStruct((nout_p, F), jnp.float32),
        mesh=_mesh(),
        compiler_params=pltpu.CompilerParams(needs_layout_passes=False),
        scratch_types=[
            pltpu.VMEM((RPW + 1, F), jnp.float32),
            pltpu.VMEM((CH,), jnp.int32),
            pltpu.VMEM((CH + GB,), jnp.int32),
            pltpu.VMEM((CH + GB,), jnp.int32),
            pltpu.VMEM((GB, F), jnp.float32),
            pltpu.VMEM((GB, F), jnp.float32),
            pltpu.SemaphoreType.DMA,
            pltpu.SemaphoreType.DMA,
        ],
    )
    def k(y_hbm, dst_hbm, out_hbm, acc, dstbuf, midbuf, lrbuf, rows0, rows1,
          gsem0, gsem1):
        w = _wid()
        lo = w * RPW

        def init_body(r, c):
            for cc in range(F // 16):
                acc[r, pl.ds(cc * 16, 16)] = jnp.full((16,), _NEG_INF, jnp.float32)
            return c
        lax.fori_loop(0, RPW + 1, init_body, 0)

        def acc_batch(bb, rref):
            def jj_body(jj, c):
                lv = lrbuf[pl.ds(bb * GB + jj * 16, 16)]
                for i in range(16):
                    lr = lv[i]
                    for cc in range(F // 16):
                        sl = pl.ds(cc * 16, 16)
                        acc[lr, sl] = jnp.maximum(acc[lr, sl],
                                                  rref[jj * 16 + i, sl])
                return c
            lax.fori_loop(0, GB // 16, jj_body, 0)

        def chunk_body(kk, _):
            pltpu.sync_copy(dst_hbm.at[pl.ds(kk * CH, CH)], dstbuf)

            def scan_body(j, ptr):
                groups = []
                for g in range(8):
                    off = j * 128 + g * 16
                    d = dstbuf[pl.ds(off, 16)]
                    lrel = d - lo
                    m = (lrel >= 0) & (lrel < RPW)
                    eid = kk * CH + off + lax.iota(jnp.int32, 16)
                    key = jnp.where(m, lrel, jnp.int32(1 << 30))
                    sk, sval = plsc.sort_key_val(key, eid)
                    cnt = plsc.all_reduce_population_count(m)
                    groups.append((sk, sval, cnt))
                for sk, sval, cnt in groups:
                    lrbuf[pl.ds(ptr, 16)] = sk
                    midbuf[pl.ds(ptr, 16)] = sval
                    ptr = ptr + cnt[0]
                return ptr

            ptr = lax.fori_loop(0, CH // 128, scan_body, jnp.int32(0))

            zz = jnp.zeros((16,), jnp.int32)
            tt = jnp.full((16,), RPW, jnp.int32)
            for q in range(GB // 16):
                midbuf[pl.ds(ptr + q * 16, 16)] = zz
                lrbuf[pl.ds(ptr + q * 16, 16)] = tt

            nsub = jnp.maximum(lax.div(ptr + (GB - 1), jnp.int32(GB)),
                               jnp.int32(1))

            pltpu.async_copy(
                y_hbm.at[midbuf.at[pl.ds(0, GB)]], rows0, gsem0)

            @pl.when(nsub > 1)
            def _():
                pltpu.async_copy(
                    y_hbm.at[midbuf.at[pl.ds(GB, GB)]], rows1, gsem1)

            pltpu.make_async_copy(y_hbm.at[pl.ds(0, GB)], rows0, gsem0).wait()
            acc_batch(0, rows0)

            @pl.when(nsub > 1)
            def _():
                pltpu.make_async_copy(
                    y_hbm.at[pl.ds(0, GB)], rows1, gsem1).wait()
                acc_batch(1, rows1)

            def tail_body(bb, c):
                pltpu.async_copy(
                    y_hbm.at[midbuf.at[pl.ds(bb * GB, GB)]], rows0, gsem0)
                pltpu.make_async_copy(
                    y_hbm.at[pl.ds(0, GB)], rows0, gsem0).wait()
                acc_batch(bb, rows0)
                return c

            lax.fori_loop(2, nsub, tail_body, 0)
            return 0

        lax.fori_loop(0, NCHUNK, chunk_body, 0)

        def fin_body(r, c):
            for cc in range(F // 16):
                sl = pl.ds(cc * 16, 16)
                v = acc[r, sl]
                acc[r, sl] = jnp.where(v > _NEG_INF, v, jnp.float32(0.0))
            return c
        lax.fori_loop(0, RPW, fin_body, 0)
        pltpu.sync_copy(acc.at[pl.ds(0, RPW)], out_hbm.at[pl.ds(lo, RPW)])

    return k(Y, dst)


def _gather_l2(X1, TP, idx1, idx2p, batch, src2p, dst2p):
    CHUNKS = E2P // 128
    KMAX = CHUNKS // NW
    UCH = N2P // 128

    out_type = [
        jax.ShapeDtypeStruct((E2P, 128), jnp.float32),
        jax.ShapeDtypeStruct((E2P, 16), jnp.float32),
        jax.ShapeDtypeStruct((N2P, 16), jnp.float32),
        jax.ShapeDtypeStruct((N2P,), jnp.int32),
    ]

    @functools.partial(
        pl.kernel,
        out_type=out_type,
        mesh=_mesh(),
        compiler_params=pltpu.CompilerParams(use_tc_tiling_on_sc=False, needs_layout_passes=False),
        scratch_types=[
            pltpu.VMEM((128,), jnp.int32),
            pltpu.VMEM((128,), jnp.int32),
            pltpu.VMEM((128,), jnp.int32),
            pltpu.VMEM((128,), jnp.int32),
            pltpu.VMEM((128,), jnp.int32),
            pltpu.VMEM((128,), jnp.int32),
            pltpu.VMEM((128, 128), jnp.float32),
            pltpu.VMEM((128, 16), jnp.float32),
            pltpu.VMEM((128, 16), jnp.float32),
            pltpu.VMEM((128, 16), jnp.float32),
            pltpu.VMEM((128,), jnp.int32),
            pltpu.SemaphoreType.DMA,
            pltpu.SemaphoreType.DMA,
            pltpu.SemaphoreType.DMA,
        ],
    )
    def k(x1_hbm, tp_hbm, idx1_hbm, idx2_hbm, batch_hbm, src2_hbm, dst2_hbm,
          h2x_hbm, h2d_hbm, u2_hbm, b2_hbm,
          sv, dv, dc, t1, csv, cdv, rx, rp1, rp2, hd, bv,
          semX, semA, semB):
        w = _wid()

        def chunk_body(kk, _):
            base = (w + kk * NW) * 128
            pltpu.sync_copy(src2_hbm.at[pl.ds(base, 128)], sv)
            pltpu.sync_copy(dst2_hbm.at[pl.ds(base, 128)], dv)
            cpX = pltpu.async_copy(x1_hbm.at[sv], rx, semX)
            for j in range(8):
                s16 = pl.ds(j * 16, 16)
                dc[s16] = jnp.minimum(dv[s16], jnp.int32(N2 - 1))
            cpS = pltpu.async_copy(idx1_hbm.at[sv], csv, semA)
            pltpu.async_copy(idx2_hbm.at[dc], t1, semB).wait()
            pltpu.async_copy(idx1_hbm.at[t1], cdv, semB).wait()
            cpS.wait()
            cpA = pltpu.async_copy(tp_hbm.at[csv], rp1, semA)
            cpB = pltpu.async_copy(tp_hbm.at[cdv], rp2, semB)
            cpX.wait()
            pltpu.sync_copy(rx, h2x_hbm.at[pl.ds(base, 128)])
            cpA.wait()
            cpB.wait()

            def sub_body(i, c):
                hd[i, :] = rp1[i, :] - rp2[i, :]
                return c
            lax.fori_loop(0, 128, sub_body, 0)
            pltpu.sync_copy(hd, h2d_hbm.at[pl.ds(base, 128)])
            return 0

        lax.fori_loop(0, KMAX, chunk_body, 0)

        def u_body(kk, _):
            chunk = w + kk * NW

            @pl.when(chunk < UCH)
            def _():
                base = chunk * 128
                pltpu.sync_copy(idx2_hbm.at[pl.ds(base, 128)], t1)
                pltpu.async_copy(idx1_hbm.at[t1], cdv, semA).wait()
                pltpu.async_copy(batch_hbm.at[cdv], dc, semA).wait()
                pltpu.async_copy(tp_hbm.at[cdv], rp1, semB).wait()
                for j in range(8):
                    s16 = pl.ds(j * 16, 16)
                    rowid = base + j * 16 + lax.iota(jnp.int32, 16)
                    bv[s16] = jnp.where(rowid < N2, dc[s16], jnp.int32(B))
                pltpu.sync_copy(rp1, u2_hbm.at[pl.ds(base, 128)])
                pltpu.sync_copy(bv, b2_hbm.at[pl.ds(base, 128)])
            return 0

        lax.fori_loop(0, 2, u_body, 0)

    return k(X1, TP, idx1, idx2p, batch, src2p, dst2p)


def _mlp1_tc(H1, W1, b1, W2, b2, W3, b3):
    BE = 2000
    G = E1 // BE

    def body(h_ref, w1, bb1, w2, bb2, w3, bb3, out_ref):
        h = h_ref[...]
        h = jnp.maximum(jnp.dot(h, w1[...], preferred_element_type=jnp.float32)
                        + bb1[0:1, :], 0.0)
        h = jnp.maximum(jnp.dot(h, w2[...], preferred_element_type=jnp.float32)
                        + bb2[0:1, :], 0.0)
        out_ref[...] = (jnp.dot(h, w3[...], preferred_element_type=jnp.float32)
                        + bb3[0:1, :])

    return pl.pallas_call(
        body,
        grid=(G,),
        in_specs=[
            pl.BlockSpec((BE, 16), lambda i: (i, 0)),
            pl.BlockSpec((16, 64), lambda i: (0, 0)),
            pl.BlockSpec((8, 64), lambda i: (0, 0)),
            pl.BlockSpec((64, 64), lambda i: (0, 0)),
            pl.BlockSpec((8, 64), lambda i: (0, 0)),
            pl.BlockSpec((64, 128), lambda i: (0, 0)),
            pl.BlockSpec((8, 128), lambda i: (0, 0)),
        ],
        out_specs=pl.BlockSpec((BE, 128), lambda i: (i, 0)),
        out_shape=jax.ShapeDtypeStruct((E1, 128), jnp.float32),
    )(H1, W1, b1, W2, b2, W3, b3)


def _mlp2_tc(H2x, H2d, Wa, Wb, b1, W2, b2, W3, b3):
    BE = 4096
    G = E2P // BE

    def body(hx_ref, hd_ref, wa, wb, bb1, w2, bb2, w3, bb3, out_ref):
        h = jnp.dot(hx_ref[...], wa[...], preferred_element_type=jnp.float32)
        h = h + jnp.dot(hd_ref[...], wb[...], preferred_element_type=jnp.float32)
        h = jnp.maximum(h + bb1[0:1, :], 0.0)
        h = jnp.maximum(jnp.dot(h, w2[...], preferred_element_type=jnp.float32)
                        + bb2[0:1, :], 0.0)
        out_ref[...] = (jnp.dot(h, w3[...], preferred_element_type=jnp.float32)
                        + bb3[0:1, :])

    return pl.pallas_call(
        body,
        grid=(G,),
        in_specs=[
            pl.BlockSpec((BE, 128), lambda i: (i, 0)),
            pl.BlockSpec((BE, 16), lambda i: (i, 0)),
            pl.BlockSpec((128, 128), lambda i: (0, 0)),
            pl.BlockSpec((16, 128), lambda i: (0, 0)),
            pl.BlockSpec((8, 128), lambda i: (0, 0)),
            pl.BlockSpec((128, 128), lambda i: (0, 0)),
            pl.BlockSpec((8, 128), lambda i: (0, 0)),
            pl.BlockSpec((128, 256), lambda i: (0, 0)),
            pl.BlockSpec((8, 256), lambda i: (0, 0)),
        ],
        out_specs=pl.BlockSpec((BE, 256), lambda i: (i, 0)),
        out_shape=jax.ShapeDtypeStruct((E2P, 256), jnp.float32),
    )(H2x, H2d, Wa, Wb, b1, W2, b2, W3, b3)


def _final_tc(X2, U2, B2r, Wa, Wb, b1, W2, b2, W3, b3,
              Wh1, bh1, Wh2, bh2, Wh3, bh3, Wp1, bp1, Wp2, bp2):
    BR = 800
    G = N2P // BR

    def body(x2_ref, u2_ref, b2_ref, wa, wb, bb1, w2, bb2, w3, bb3,
             wh1, bbh1, wh2, bbh2, wh3, bbh3, wp1, bbp1, wp2, bbp2,
             out_ref, acc):
        step = pl.program_id(0)

        @pl.when(step == 0)
        def _():
            acc[...] = jnp.full((B, 1024), _NEG_INF, jnp.float32)

        h = jnp.dot(x2_ref[...], wa[...], preferred_element_type=jnp.float32)
        h = h + jnp.dot(u2_ref[...], wb[...], preferred_element_type=jnp.float32)
        h = jnp.maximum(h + bb1[0:1, :], 0.0)
        h = jnp.maximum(jnp.dot(h, w2[...], preferred_element_type=jnp.float32)
                        + bb2[0:1, :], 0.0)
        h = jnp.dot(h, w3[...], preferred_element_type=jnp.float32) + bb3[0:1, :]
        bid = b2_ref[...][:, 0:1]
        parts = []
        for bb in range(B):
            hm = jnp.where(bid == bb, h, _NEG_INF)
            parts.append(jnp.max(hm, axis=0, keepdims=True))
        acc[...] = jnp.maximum(acc[...], jnp.concatenate(parts, axis=0))

        @pl.when(step == G - 1)
        def _():
            xg = acc[...]
            xg = jnp.where(xg > _NEG_INF, xg, 0.0)
            f = jnp.maximum(jnp.dot(xg, wh1[...], preferred_element_type=jnp.float32)
                            + bbh1[0:1, :], 0.0)
            f = jnp.maximum(jnp.dot(f, wh2[...], preferred_element_type=jnp.float32)
                            + bbh2[0:1, :], 0.0)
            f = jnp.dot(f, wh3[...], preferred_element_type=jnp.float32) + bbh3[0:1, :]
            f = jnp.maximum(jnp.dot(f, wp1[...], preferred_element_type=jnp.float32)
                            + bbp1[0:1, :], 0.0)
            out_ref[...] = (jnp.dot(f, wp2[...], preferred_element_type=jnp.float32)
                            + bbp2[0:1, :])

    full = lambda s: pl.BlockSpec(s, lambda i: (0, 0))
    return pl.pallas_call(
        body,
        grid=(G,),
        in_specs=[
            pl.BlockSpec((BR, 256), lambda i: (i, 0)),
            pl.BlockSpec((BR, 16), lambda i: (i, 0)),
            pl.BlockSpec((BR, 128), lambda i: (i, 0)),
            full((256, 256)), full((16, 256)), full((8, 256)),
            full((256, 512)), full((8, 512)),
            full((512, 1024)), full((8, 1024)),
            full((1024, 512)), full((8, 512)),
            full((512, 256)), full((8, 256)),
            full((256, 32)), full((8, 32)),
            full((32, 32)), full((8, 32)),
            full((32, 8)), full((8, 8)),
        ],
        out_specs=pl.BlockSpec((B, 8), lambda i: (0, 0)),
        out_shape=jax.ShapeDtypeStruct((B, 8), jnp.float32),
        scratch_shapes=[pltpu.VMEM((B, 1024), jnp.float32)],
    )(X2, U2, B2r, Wa, Wb, b1, W2, b2, W3, b3,
      Wh1, bh1, Wh2, bh2, Wh3, bh3, Wp1, bp1, Wp2, bp2)


def kernel(x, pos, batch, idx1, src1, dst1, idx2, src2, dst2,
           params1, params2, params3, params_head, params_policy):
    f32 = jnp.float32
    (W11, b11), (W12, b12), (W13, b13) = params1
    (W21, b21), (W22, b22), (W23, b23) = params2
    (W31, b31), (W32, b32), (W33, b33) = params3
    (Wh1, bh1), (Wh2, bh2), (Wh3, bh3) = params_head
    (Wp1, bp1), (Wp2, bp2) = params_policy

    T = jnp.concatenate([x, pos, jnp.zeros((N, 1), f32)], axis=1)
    TP = jnp.concatenate([jnp.zeros((N, 12), f32), pos,
                          jnp.zeros((N, 1), f32)], axis=1)

    H1 = _gather_h1(T, TP, idx1, src1, dst1)
    W1p = jnp.concatenate([W11, jnp.zeros((1, 64), f32)], axis=0)
    Y1 = _mlp1_tc(H1, W1p, _tile8(b11), W12, _tile8(b12), W13, _tile8(b13))
    dst1p = jnp.concatenate([dst1, jnp.full((409600 - E1,), PAD_DST, jnp.int32)])
    X1 = _scatter_max(Y1, dst1p, N1P, 128, 200)

    idx2p = jnp.concatenate([idx2, jnp.zeros((N2P - N2,), jnp.int32)])
    src2p = jnp.concatenate([src2, jnp.zeros((E2P - E2,), jnp.int32)])
    dst2p = jnp.concatenate([dst2, jnp.full((E2P - E2,), PAD_DST, jnp.int32)])
    H2x, H2d, U2, B2 = _gather_l2(X1, TP, idx1, idx2p, batch, src2p, dst2p)

    W2a = W21[:128, :]
    W2b = jnp.zeros((16, 128), f32).at[12:15, :].set(W21[128:131, :])
    Y2 = _mlp2_tc(H2x, H2d, W2a, W2b, _tile8(b21), W22, _tile8(b22),
                  W23, _tile8(b23))
    X2 = _scatter_max(Y2, dst2p, N2P, 256, 98)

    W3a = W31[:256, :]
    W3b = jnp.zeros((16, 256), f32).at[12:15, :].set(W31[256:259, :])
    B2r = jnp.broadcast_to(B2[:, None], (N2P, 128))
    logits = _final_tc(X2, U2, B2r, W3a, W3b, _tile8(b31), W32, _tile8(b32),
                       W33, _tile8(b33), Wh1, _tile8(bh1), Wh2, _tile8(bh2),
                       Wh3, _tile8(bh3), Wp1, _tile8(bp1), Wp2, _tile8(bp2))
    return logits


# speedup vs baseline: 3.5130x; 1.8099x over previous
import functools

import jax
import jax.numpy as jnp
from jax import lax
from jax.experimental import pallas as pl
from jax.experimental.pallas import tpu as pltpu
from jax.experimental.pallas import tpu_sc as plsc

N = 50000
N1 = 25000
N2 = 6250
E1 = 400000
E2 = 200000
B = 16

NC = 2
NS = 16
NW = NC * NS

N1P = 25088
N2P = 6400
E2P = 200704
PAD_DST = 1 << 20

_NEG_INF = float("-inf")


def _mesh():
    return plsc.VectorSubcoreMesh(core_axis_name="c", subcore_axis_name="s")


def _wid():
    return lax.axis_index("s") * NC + lax.axis_index("c")


def _tile8(b):
    return jnp.broadcast_to(b[None, :], (8, b.shape[0]))


def _gather_h1(T, TP, idx1, src1, dst1):
    CHUNKS = E1 // 128
    KMAX = (CHUNKS + NW - 1) // NW

    @functools.partial(
        pl.kernel,
        out_type=jax.ShapeDtypeStruct((E1, 16), jnp.float32),
        mesh=_mesh(),
        compiler_params=pltpu.CompilerParams(use_tc_tiling_on_sc=False, needs_layout_passes=False),
        scratch_types=[
            pltpu.VMEM((128,), jnp.int32),
            pltpu.VMEM((128,), jnp.int32),
            pltpu.VMEM((128,), jnp.int32),
            pltpu.VMEM((128, 16), jnp.float32),
            pltpu.VMEM((128, 16), jnp.float32),
            pltpu.VMEM((128, 16), jnp.float32),
            pltpu.SemaphoreType.DMA,
            pltpu.SemaphoreType.DMA,
        ],
    )
    def k(T_hbm, TP_hbm, idx1_hbm, src1_hbm, dst1_hbm, h1_hbm,
          sv, dv, ds_v, ra, rb, hb, semA, semB):
        w = _wid()

        def chunk_body(kk, _):
            chunk = w + kk * NW

            @pl.when(chunk < CHUNKS)
            def _():
                base = chunk * 128
                pltpu.sync_copy(src1_hbm.at[pl.ds(base, 128)], sv)
                pltpu.sync_copy(dst1_hbm.at[pl.ds(base, 128)], dv)
                pltpu.async_copy(idx1_hbm.at[dv], ds_v, semA).wait()
                cpA = pltpu.async_copy(T_hbm.at[sv], ra, semA)
                cpB = pltpu.async_copy(TP_hbm.at[ds_v], rb, semB)
                cpA.wait()
                cpB.wait()

                def sub_body(i, c):
                    hb[i, :] = ra[i, :] - rb[i, :]
                    return c
                lax.fori_loop(0, 128, sub_body, 0)
                pltpu.sync_copy(hb, h1_hbm.at[pl.ds(base, 128)])
            return 0

        lax.fori_loop(0, KMAX, chunk_body, 0)

    return k(T, TP, idx1, src1, dst1)


def _scatter_max(Y, dst, nout_p, F, NCHUNK):
    RPW = nout_p // NW
    GB = 32
    CH = 2048

    @functools.partial(
        pl.kernel,
        out_type=jax.ShapeDtypeStruct((nout_p, F), jnp.float32),
        mesh=_mesh(),
        compiler_params=pltpu.CompilerParams(needs_layout_passes=False),
        scratch_types=[
            pltpu.VMEM((RPW + 1, F), jnp.float32),
            pltpu.VMEM((CH,), jnp.int32),
            pltpu.VMEM((CH + GB,), jnp.int32),
            pltpu.VMEM((CH + GB,), jnp.int32),
            pltpu.VMEM((GB, F), jnp.float32),
            pltpu.VMEM((GB, F), jnp.float32),
            pltpu.SemaphoreType.DMA,
            pltpu.SemaphoreType.DMA,
        ],
    )
    def k(y_hbm, dst_hbm, out_hbm, acc, dstbuf, midbuf, lrbuf, rows0, rows1,
          gsem0, gsem1):
        w = _wid()
        lo = w * RPW

        def init_body(r, c):
            for cc in range(F // 16):
                acc[r, pl.ds(cc * 16, 16)] = jnp.full((16,), _NEG_INF, jnp.float32)
            return c
        lax.fori_loop(0, RPW + 1, init_body, 0)

        def acc_batch(bb, rref):
            def jj_body(jj, c):
                lv = lrbuf[pl.ds(bb * GB + jj * 16, 16)]
                for i in range(16):
                    lr = lv[i]
                    for cc in range(F // 16):
                        sl = pl.ds(cc * 16, 16)
                        acc[lr, sl] = jnp.maximum(acc[lr, sl],
                                                  rref[jj * 16 + i, sl])
                return c
            lax.fori_loop(0, GB // 16, jj_body, 0)

        def chunk_body(kk, _):
            pltpu.sync_copy(dst_hbm.at[pl.ds(kk * CH, CH)], dstbuf)

            def scan_body(j, ptr):
                groups = []
                for g in range(8):
                    off = j * 128 + g * 16
                    d = dstbuf[pl.ds(off, 16)]
                    lrel = d - lo
                    m = (lrel >= 0) & (lrel < RPW)
                    eid = kk * CH + off + lax.iota(jnp.int32, 16)
                    key = jnp.where(m, lrel, jnp.int32(1 << 30))
                    sk, sval = plsc.sort_key_val(key, eid)
                    cnt = plsc.all_reduce_population_count(m)
                    groups.append((sk, sval, cnt))
                for sk, sval, cnt in groups:
                    lrbuf[pl.ds(ptr, 16)] = sk
                    midbuf[pl.ds(ptr, 16)] = sval
                    ptr = ptr + cnt[0]
                return ptr

            ptr = lax.fori_loop(0, CH // 128, scan_body, jnp.int32(0))

            zz = jnp.zeros((16,), jnp.int32)
            tt = jnp.full((16,), RPW, jnp.int32)
            for q in range(GB // 16):
                midbuf[pl.ds(ptr + q * 16, 16)] = zz
                lrbuf[pl.ds(ptr + q * 16, 16)] = tt

            nsub = jnp.maximum(lax.div(ptr + (GB - 1), jnp.int32(GB)),
                               jnp.int32(1))

            pltpu.async_copy(
                y_hbm.at[midbuf.at[pl.ds(0, GB)]], rows0, gsem0)

            @pl.when(nsub > 1)
            def _():
                pltpu.async_copy(
                    y_hbm.at[midbuf.at[pl.ds(GB, GB)]], rows1, gsem1)

            pltpu.make_async_copy(y_hbm.at[pl.ds(0, GB)], rows0, gsem0).wait()
            acc_batch(0, rows0)

            @pl.when(nsub > 1)
            def _():
                pltpu.make_async_copy(
                    y_hbm.at[pl.ds(0, GB)], rows1, gsem1).wait()
                acc_batch(1, rows1)

            def tail_body(bb, c):
                pltpu.async_copy(
                    y_hbm.at[midbuf.at[pl.ds(bb * GB, GB)]], rows0, gsem0)
                pltpu.make_async_copy(
                    y_hbm.at[pl.ds(0, GB)], rows0, gsem0).wait()
                acc_batch(bb, rows0)
                return c

            lax.fori_loop(2, nsub, tail_body, 0)
            return 0

        lax.fori_loop(0, NCHUNK, chunk_body, 0)

        def fin_body(r, c):
            for cc in range(F // 16):
                sl = pl.ds(cc * 16, 16)
                v = acc[r, sl]
                acc[r, sl] = jnp.where(v > _NEG_INF, v, jnp.float32(0.0))
            return c
        lax.fori_loop(0, RPW, fin_body, 0)
        pltpu.sync_copy(acc.at[pl.ds(0, RPW)], out_hbm.at[pl.ds(lo, RPW)])

    return k(Y, dst)


def _gather_l2(X1, TP, idx1, idx2p, batch, src2p, dst2p):
    CHUNKS = E2P // 128
    KMAX = CHUNKS // NW
    UCH = N2P // 128

    out_type = [
        jax.ShapeDtypeStruct((E2P, 128), jnp.float32),
        jax.ShapeDtypeStruct((E2P, 16), jnp.float32),
        jax.ShapeDtypeStruct((N2P, 16), jnp.float32),
        jax.ShapeDtypeStruct((N2P,), jnp.int32),
    ]

    @functools.partial(
        pl.kernel,
        out_type=out_type,
        mesh=_mesh(),
        compiler_params=pltpu.CompilerParams(use_tc_tiling_on_sc=False, needs_layout_passes=False),
        scratch_types=[
            pltpu.VMEM((128,), jnp.int32),
            pltpu.VMEM((128,), jnp.int32),
            pltpu.VMEM((128,), jnp.int32),
            pltpu.VMEM((128,), jnp.int32),
            pltpu.VMEM((128,), jnp.int32),
            pltpu.VMEM((128,), jnp.int32),
            pltpu.VMEM((128, 128), jnp.float32),
            pltpu.VMEM((128, 16), jnp.float32),
            pltpu.VMEM((128, 16), jnp.float32),
            pltpu.VMEM((128, 16), jnp.float32),
            pltpu.VMEM((128,), jnp.int32),
            pltpu.SemaphoreType.DMA,
            pltpu.SemaphoreType.DMA,
            pltpu.SemaphoreType.DMA,
        ],
    )
    def k(x1_hbm, tp_hbm, idx1_hbm, idx2_hbm, batch_hbm, src2_hbm, dst2_hbm,
          h2x_hbm, h2d_hbm, u2_hbm, b2_hbm,
          sv, dv, dc, t1, csv, cdv, rx, rp1, rp2, hd, bv,
          semX, semA, semB):
        w = _wid()

        def chunk_body(kk, _):
            base = (w + kk * NW) * 128
            pltpu.sync_copy(src2_hbm.at[pl.ds(base, 128)], sv)
            pltpu.sync_copy(dst2_hbm.at[pl.ds(base, 128)], dv)
            cpX = pltpu.async_copy(x1_hbm.at[sv], rx, semX)
            for j in range(8):
                s16 = pl.ds(j * 16, 16)
                dc[s16] = jnp.minimum(dv[s16], jnp.int32(N2 - 1))
            cpS = pltpu.async_copy(idx1_hbm.at[sv], csv, semA)
            pltpu.async_copy(idx2_hbm.at[dc], t1, semB).wait()
            pltpu.async_copy(idx1_hbm.at[t1], cdv, semB).wait()
            cpS.wait()
            cpA = pltpu.async_copy(tp_hbm.at[csv], rp1, semA)
            cpB = pltpu.async_copy(tp_hbm.at[cdv], rp2, semB)
            cpX.wait()
            pltpu.sync_copy(rx, h2x_hbm.at[pl.ds(base, 128)])
            cpA.wait()
            cpB.wait()

            def sub_body(i, c):
                hd[i, :] = rp1[i, :] - rp2[i, :]
                return c
            lax.fori_loop(0, 128, sub_body, 0)
            pltpu.sync_copy(hd, h2d_hbm.at[pl.ds(base, 128)])
            return 0

        lax.fori_loop(0, KMAX, chunk_body, 0)

        def u_body(kk, _):
            chunk = w + kk * NW

            @pl.when(chunk < UCH)
            def _():
                base = chunk * 128
                pltpu.sync_copy(idx2_hbm.at[pl.ds(base, 128)], t1)
                pltpu.async_copy(idx1_hbm.at[t1], cdv, semA).wait()
                pltpu.async_copy(batch_hbm.at[cdv], dc, semA).wait()
                pltpu.async_copy(tp_hbm.at[cdv], rp1, semB).wait()
                for j in range(8):
                    s16 = pl.ds(j * 16, 16)
                    rowid = base + j * 16 + lax.iota(jnp.int32, 16)
                    bv[s16] = jnp.where(rowid < N2, dc[s16], jnp.int32(B))
                pltpu.sync_copy(rp1, u2_hbm.at[pl.ds(base, 128)])
                pltpu.sync_copy(bv, b2_hbm.at[pl.ds(base, 128)])
            return 0

        lax.fori_loop(0, 2, u_body, 0)

    return k(X1, TP, idx1, idx2p, batch, src2p, dst2p)


def _mlp1_tc(H1, W1, b1, W2, b2, W3, b3):
    BE = 2000
    G = E1 // BE

    def body(h_ref, w1, bb1, w2, bb2, w3, bb3, out_ref):
        h = h_ref[...]
        h = jnp.maximum(jnp.dot(h, w1[...], preferred_element_type=jnp.float32)
                        + bb1[0:1, :], 0.0)
        h = jnp.maximum(jnp.dot(h, w2[...], preferred_element_type=jnp.float32)
                        + bb2[0:1, :], 0.0)
        out_ref[...] = (jnp.dot(h, w3[...], preferred_element_type=jnp.float32)
                        + bb3[0:1, :])

    return pl.pallas_call(
        body,
        grid=(G,),
        in_specs=[
            pl.BlockSpec((BE, 16), lambda i: (i, 0)),
            pl.BlockSpec((16, 64), lambda i: (0, 0)),
            pl.BlockSpec((8, 64), lambda i: (0, 0)),
            pl.BlockSpec((64, 64), lambda i: (0, 0)),
            pl.BlockSpec((8, 64), lambda i: (0, 0)),
            pl.BlockSpec((64, 128), lambda i: (0, 0)),
            pl.BlockSpec((8, 128), lambda i: (0, 0)),
        ],
        out_specs=pl.BlockSpec((BE, 128), lambda i: (i, 0)),
        out_shape=jax.ShapeDtypeStruct((E1, 128), jnp.float32),
    )(H1, W1, b1, W2, b2, W3, b3)


def _mlp2_tc(H2x, H2d, Wa, Wb, b1, W2, b2, W3, b3):
    BE = 4096
    G = E2P // BE

    def body(hx_ref, hd_ref, wa, wb, bb1, w2, bb2, w3, bb3, out_ref):
        h = jnp.dot(hx_ref[...], wa[...], preferred_element_type=jnp.float32)
        h = h + jnp.dot(hd_ref[...], wb[...], preferred_element_type=jnp.float32)
        h = jnp.maximum(h + bb1[0:1, :], 0.0)
        h = jnp.maximum(jnp.dot(h, w2[...], preferred_element_type=jnp.float32)
                        + bb2[0:1, :], 0.0)
        out_ref[...] = (jnp.dot(h, w3[...], preferred_element_type=jnp.float32)
                        + bb3[0:1, :])

    return pl.pallas_call(
        body,
        grid=(G,),
        in_specs=[
            pl.BlockSpec((BE, 128), lambda i: (i, 0)),
            pl.BlockSpec((BE, 16), lambda i: (i, 0)),
            pl.BlockSpec((128, 128), lambda i: (0, 0)),
            pl.BlockSpec((16, 128), lambda i: (0, 0)),
            pl.BlockSpec((8, 128), lambda i: (0, 0)),
            pl.BlockSpec((128, 128), lambda i: (0, 0)),
            pl.BlockSpec((8, 128), lambda i: (0, 0)),
            pl.BlockSpec((128, 256), lambda i: (0, 0)),
            pl.BlockSpec((8, 256), lambda i: (0, 0)),
        ],
        out_specs=pl.BlockSpec((BE, 256), lambda i: (i, 0)),
        out_shape=jax.ShapeDtypeStruct((E2P, 256), jnp.float32),
    )(H2x, H2d, Wa, Wb, b1, W2, b2, W3, b3)


def _final_tc(X2, U2, B2r, Wa, Wb, b1, W2, b2, W3, b3,
              Wh1, bh1, Wh2, bh2, Wh3, bh3, Wp1, bp1, Wp2, bp2):
    BR = 800
    G = N2P // BR

    def body(x2_ref, u2_ref, b2_ref, wa, wb, bb1, w2, bb2, w3, bb3,
             wh1, bbh1, wh2, bbh2, wh3, bbh3, wp1, bbp1, wp2, bbp2,
             out_ref, acc):
        step = pl.program_id(0)

        @pl.when(step == 0)
        def _():
            acc[...] = jnp.full((B, 1024), _NEG_INF, jnp.float32)

        h = jnp.dot(x2_ref[...], wa[...], preferred_element_type=jnp.float32)
        h = h + jnp.dot(u2_ref[...], wb[...], preferred_element_type=jnp.float32)
        h = jnp.maximum(h + bb1[0:1, :], 0.0)
        h = jnp.maximum(jnp.dot(h, w2[...], preferred_element_type=jnp.float32)
                        + bb2[0:1, :], 0.0)
        h = jnp.dot(h, w3[...], preferred_element_type=jnp.float32) + bb3[0:1, :]
        bid = b2_ref[...][:, 0:1]
        parts = []
        for bb in range(B):
            hm = jnp.where(bid == bb, h, _NEG_INF)
            parts.append(jnp.max(hm, axis=0, keepdims=True))
        acc[...] = jnp.maximum(acc[...], jnp.concatenate(parts, axis=0))

        @pl.when(step == G - 1)
        def _():
            xg = acc[...]
            xg = jnp.where(xg > _NEG_INF, xg, 0.0)
            f = jnp.maximum(jnp.dot(xg, wh1[...], preferred_element_type=jnp.float32)
                            + bbh1[0:1, :], 0.0)
            f = jnp.maximum(jnp.dot(f, wh2[...], preferred_element_type=jnp.float32)
                            + bbh2[0:1, :], 0.0)
            f = jnp.dot(f, wh3[...], preferred_element_type=jnp.float32) + bbh3[0:1, :]
            f = jnp.maximum(jnp.dot(f, wp1[...], preferred_element_type=jnp.float32)
                            + bbp1[0:1, :], 0.0)
            out_ref[...] = (jnp.dot(f, wp2[...], preferred_element_type=jnp.float32)
                            + bbp2[0:1, :])

    full = lambda s: pl.BlockSpec(s, lambda i: (0, 0))
    return pl.pallas_call(
        body,
        grid=(G,),
        in_specs=[
            pl.BlockSpec((BR, 256), lambda i: (i, 0)),
            pl.BlockSpec((BR, 16), lambda i: (i, 0)),
            pl.BlockSpec((BR, 128), lambda i: (i, 0)),
            full((256, 256)), full((16, 256)), full((8, 256)),
            full((256, 512)), full((8, 512)),
            full((512, 1024)), full((8, 1024)),
            full((1024, 512)), full((8, 512)),
            full((512, 256)), full((8, 256)),
            full((256, 32)), full((8, 32)),
            full((32, 32)), full((8, 32)),
            full((32, 8)), full((8, 8)),
        ],
        out_specs=pl.BlockSpec((B, 8), lambda i: (0, 0)),
        out_shape=jax.ShapeDtypeStruct((B, 8), jnp.float32),
        scratch_shapes=[pltpu.VMEM((B, 1024), jnp.float32)],
    )(X2, U2, B2r, Wa, Wb, b1, W2, b2, W3, b3,
      Wh1, bh1, Wh2, bh2, Wh3, bh3, Wp1, bp1, Wp2, bp2)


def kernel(x, pos, batch, idx1, src1, dst1, idx2, src2, dst2,
           params1, params2, params3, params_head, params_policy):
    f32 = jnp.float32
    (W11, b11), (W12, b12), (W13, b13) = params1
    (W21, b21), (W22, b22), (W23, b23) = params2
    (W31, b31), (W32, b32), (W33, b33) = params3
    (Wh1, bh1), (Wh2, bh2), (Wh3, bh3) = params_head
    (Wp1, bp1), (Wp2, bp2) = params_policy

    T = jnp.concatenate([x, pos, jnp.zeros((N, 1), f32)], axis=1)
    TP = jnp.concatenate([jnp.zeros((N, 12), f32), pos,
                          jnp.zeros((N, 1), f32)], axis=1)

    H1 = _gather_h1(T, TP, idx1, src1, dst1)
    W1p = jnp.concatenate([W11, jnp.zeros((1, 64), f32)], axis=0)
    Y1 = _mlp1_tc(H1, W1p, _tile8(b11), W12, _tile8(b12), W13, _tile8(b13))
    dst1p = jnp.concatenate([dst1, jnp.full((409600 - E1,), PAD_DST, jnp.int32)])
    X1 = _scatter_max(Y1, dst1p, N1P, 128, 200)

    idx2p = jnp.concatenate([idx2, jnp.zeros((N2P - N2,), jnp.int32)])
    src2p = jnp.concatenate([src2, jnp.zeros((E2P - E2,), jnp.int32)])
    dst2p = jnp.concatenate([dst2, jnp.full((E2P - E2,), PAD_DST, jnp.int32)])
    H2x, H2d, U2, B2 = _gather_l2(X1, TP, idx1, idx2p, batch, src2p, dst2p)

    W2a = W21[:128, :]
    W2b = jnp.zeros((16, 128), f32).at[12:15, :].set(W21[128:131, :])
    Y2 = _mlp2_tc(H2x, H2d, W2a, W2b, _tile8(b21), W22, _tile8(b22),
                  W23, _tile8(b23))
    X2 = _scatter_max(Y2, dst2p, N2P, 256, 98)

    W3a = W31[:256, :]
    W3b = jnp.zeros((16, 256), f32).at[12:15, :].set(W31[256:259, :])
    B2r = jnp.broadcast_to(B2[:, None], (N2P, 128))
    logits = _final_tc(X2, U2, B2r, W3a, W3b, _tile8(b31), W32, _tile8(b32),
                       W33, _tile8(b33), Wh1, _tile8(bh1), Wh2, _tile8(bh2),
                       Wh3, _tile8(bh3), Wp1, _tile8(bp1), Wp2, _tile8(bp2))
    return logits


# speedup vs baseline: 5.6944x; 1.6209x over previous
import functools

import jax
import jax.numpy as jnp
from jax import lax
from jax.experimental import pallas as pl
from jax.experimental.pallas import tpu as pltpu
from jax.experimental.pallas import tpu_sc as plsc

N = 50000
N1 = 25000
N2 = 6250
E1 = 400000
E2 = 200000
B = 16

NC = 2
NS = 16
NW = NC * NS

N1P = 25088
N2P = 6400
E2P = 200704
PAD_DST = 1 << 20

_NEG_INF = float("-inf")


def _mesh():
    return plsc.VectorSubcoreMesh(core_axis_name="c", subcore_axis_name="s")


def _wid():
    return lax.axis_index("s") * NC + lax.axis_index("c")


def _tile8(b):
    return jnp.broadcast_to(b[None, :], (8, b.shape[0]))


def _gather_h1(T, TP, idx1, src1, dst1):
    CHUNKS = E1 // 128
    KMAX = (CHUNKS + NW - 1) // NW

    @functools.partial(
        pl.kernel,
        out_type=jax.ShapeDtypeStruct((E1, 16), jnp.float32),
        mesh=_mesh(),
        compiler_params=pltpu.CompilerParams(use_tc_tiling_on_sc=False, needs_layout_passes=False),
        scratch_types=[
            pltpu.VMEM((128,), jnp.int32),
            pltpu.VMEM((128,), jnp.int32),
            pltpu.VMEM((128,), jnp.int32),
            pltpu.VMEM((128, 16), jnp.float32),
            pltpu.VMEM((128, 16), jnp.float32),
            pltpu.VMEM((128, 16), jnp.float32),
            pltpu.SemaphoreType.DMA,
            pltpu.SemaphoreType.DMA,
        ],
    )
    def k(T_hbm, TP_hbm, idx1_hbm, src1_hbm, dst1_hbm, h1_hbm,
          sv, dv, ds_v, ra, rb, hb, semA, semB):
        w = _wid()

        def chunk_body(kk, _):
            chunk = w + kk * NW

            @pl.when(chunk < CHUNKS)
            def _():
                base = chunk * 128
                pltpu.sync_copy(src1_hbm.at[pl.ds(base, 128)], sv)
                pltpu.sync_copy(dst1_hbm.at[pl.ds(base, 128)], dv)
                pltpu.async_copy(idx1_hbm.at[dv], ds_v, semA).wait()
                cpA = pltpu.async_copy(T_hbm.at[sv], ra, semA)
                cpB = pltpu.async_copy(TP_hbm.at[ds_v], rb, semB)
                cpA.wait()
                cpB.wait()

                def sub_body(i, c):
                    hb[i, :] = ra[i, :] - rb[i, :]
                    return c
                lax.fori_loop(0, 128, sub_body, 0)
                pltpu.sync_copy(hb, h1_hbm.at[pl.ds(base, 128)])
            return 0

        lax.fori_loop(0, KMAX, chunk_body, 0)

    return k(T, TP, idx1, src1, dst1)


def _scatter_max(Y, dst, nout_p, F, NCHUNK):
    RPW = nout_p // NW
    GB = 16
    CH = 2048

    @functools.partial(
        pl.kernel,
        out_type=jax.ShapeDtypeStruct((nout_p, F), jnp.float32),
        mesh=_mesh(),
        compiler_params=pltpu.CompilerParams(needs_layout_passes=False),
        scratch_types=[
            pltpu.VMEM((RPW + 1, F), jnp.float32),
            pltpu.VMEM((CH,), jnp.int32),
            pltpu.VMEM((CH + GB,), jnp.int32),
            pltpu.VMEM((CH + GB,), jnp.int32),
            pltpu.VMEM((GB, F), jnp.float32),
            pltpu.VMEM((GB, F), jnp.float32),
            pltpu.SemaphoreType.DMA,
            pltpu.SemaphoreType.DMA,
        ],
    )
    def k(y_hbm, dst_hbm, out_hbm, acc, dstbuf, midbuf, lrbuf, rows0, rows1,
          gsem0, gsem1):
        w = _wid()
        lo = w * RPW

        def init_body(r, c):
            for cc in range(F // 16):
                acc[r, pl.ds(cc * 16, 16)] = jnp.full((16,), _NEG_INF, jnp.float32)
            return c
        lax.fori_loop(0, RPW + 1, init_body, 0)

        def acc_batch(bb, rref):
            def jj_body(jj, c):
                lv = lrbuf[pl.ds(bb * GB + jj * 16, 16)]
                for i in range(16):
                    lr = lv[i]
                    for cc in range(F // 16):
                        sl = pl.ds(cc * 16, 16)
                        acc[lr, sl] = jnp.maximum(acc[lr, sl],
                                                  rref[jj * 16 + i, sl])
                return c
            lax.fori_loop(0, GB // 16, jj_body, 0)

        def chunk_body(kk, _):
            pltpu.sync_copy(dst_hbm.at[pl.ds(kk * CH, CH)], dstbuf)

            def scan_body(j, ptr):
                groups = []
                for g in range(8):
                    off = j * 128 + g * 16
                    d = dstbuf[pl.ds(off, 16)]
                    lrel = d - lo
                    m = (lrel >= 0) & (lrel < RPW)
                    eid = kk * CH + off + lax.iota(jnp.int32, 16)
                    key = jnp.where(m, lrel, jnp.int32(1 << 30))
                    sk, sval = plsc.sort_key_val(key, eid)
                    cnt = plsc.all_reduce_population_count(m)
                    groups.append((sk, sval, cnt))
                for sk, sval, cnt in groups:
                    lrbuf[pl.ds(ptr, 16)] = sk
                    midbuf[pl.ds(ptr, 16)] = sval
                    ptr = ptr + cnt[0]
                return ptr

            ptr = lax.fori_loop(0, CH // 128, scan_body, jnp.int32(0))

            zz = jnp.zeros((16,), jnp.int32)
            tt = jnp.full((16,), RPW, jnp.int32)
            for q in range(GB // 16):
                midbuf[pl.ds(ptr + q * 16, 16)] = zz
                lrbuf[pl.ds(ptr + q * 16, 16)] = tt

            nsub = jnp.maximum(lax.div(ptr + (GB - 1), jnp.int32(GB)),
                               jnp.int32(1))

            pltpu.async_copy(
                y_hbm.at[midbuf.at[pl.ds(0, GB)]], rows0, gsem0)

            @pl.when(nsub > 1)
            def _():
                pltpu.async_copy(
                    y_hbm.at[midbuf.at[pl.ds(GB, GB)]], rows1, gsem1)

            pltpu.make_async_copy(y_hbm.at[pl.ds(0, GB)], rows0, gsem0).wait()
            acc_batch(0, rows0)

            @pl.when(nsub > 1)
            def _():
                pltpu.make_async_copy(
                    y_hbm.at[pl.ds(0, GB)], rows1, gsem1).wait()
                acc_batch(1, rows1)

            def tail_body(bb, c):
                pltpu.async_copy(
                    y_hbm.at[midbuf.at[pl.ds(bb * GB, GB)]], rows0, gsem0)
                pltpu.make_async_copy(
                    y_hbm.at[pl.ds(0, GB)], rows0, gsem0).wait()
                acc_batch(bb, rows0)
                return c

            lax.fori_loop(2, nsub, tail_body, 0)
            return 0

        lax.fori_loop(0, NCHUNK, chunk_body, 0)

        def fin_body(r, c):
            for cc in range(F // 16):
                sl = pl.ds(cc * 16, 16)
                v = acc[r, sl]
                acc[r, sl] = jnp.where(v > _NEG_INF, v, jnp.float32(0.0))
            return c
        lax.fori_loop(0, RPW, fin_body, 0)
        pltpu.sync_copy(acc.at[pl.ds(0, RPW)], out_hbm.at[pl.ds(lo, RPW)])

    return k(Y, dst)


def _gather_l2(X1, TP, idx1, idx2p, batch, src2p, dst2p):
    CHUNKS = E2P // 128
    KMAX = CHUNKS // NW
    UCH = N2P // 128

    out_type = [
        jax.ShapeDtypeStruct((E2P, 128), jnp.float32),
        jax.ShapeDtypeStruct((E2P, 16), jnp.float32),
        jax.ShapeDtypeStruct((N2P, 16), jnp.float32),
        jax.ShapeDtypeStruct((N2P,), jnp.int32),
    ]

    @functools.partial(
        pl.kernel,
        out_type=out_type,
        mesh=_mesh(),
        compiler_params=pltpu.CompilerParams(use_tc_tiling_on_sc=False, needs_layout_passes=False),
        scratch_types=[
            pltpu.VMEM((128,), jnp.int32),
            pltpu.VMEM((128,), jnp.int32),
            pltpu.VMEM((128,), jnp.int32),
            pltpu.VMEM((128,), jnp.int32),
            pltpu.VMEM((128,), jnp.int32),
            pltpu.VMEM((128,), jnp.int32),
            pltpu.VMEM((128, 128), jnp.float32),
            pltpu.VMEM((128, 16), jnp.float32),
            pltpu.VMEM((128, 16), jnp.float32),
            pltpu.VMEM((128, 16), jnp.float32),
            pltpu.VMEM((128,), jnp.int32),
            pltpu.SemaphoreType.DMA,
            pltpu.SemaphoreType.DMA,
            pltpu.SemaphoreType.DMA,
        ],
    )
    def k(x1_hbm, tp_hbm, idx1_hbm, idx2_hbm, batch_hbm, src2_hbm, dst2_hbm,
          h2x_hbm, h2d_hbm, u2_hbm, b2_hbm,
          sv, dv, dc, t1, csv, cdv, rx, rp1, rp2, hd, bv,
          semX, semA, semB):
        w = _wid()

        def chunk_body(kk, _):
            base = (w + kk * NW) * 128
            pltpu.sync_copy(src2_hbm.at[pl.ds(base, 128)], sv)
            pltpu.sync_copy(dst2_hbm.at[pl.ds(base, 128)], dv)
            cpX = pltpu.async_copy(x1_hbm.at[sv], rx, semX)
            for j in range(8):
                s16 = pl.ds(j * 16, 16)
                dc[s16] = jnp.minimum(dv[s16], jnp.int32(N2 - 1))
            cpS = pltpu.async_copy(idx1_hbm.at[sv], csv, semA)
            pltpu.async_copy(idx2_hbm.at[dc], t1, semB).wait()
            pltpu.async_copy(idx1_hbm.at[t1], cdv, semB).wait()
            cpS.wait()
            cpA = pltpu.async_copy(tp_hbm.at[csv], rp1, semA)
            cpB = pltpu.async_copy(tp_hbm.at[cdv], rp2, semB)
            cpX.wait()
            pltpu.sync_copy(rx, h2x_hbm.at[pl.ds(base, 128)])
            cpA.wait()
            cpB.wait()

            def sub_body(i, c):
                hd[i, :] = rp1[i, :] - rp2[i, :]
                return c
            lax.fori_loop(0, 128, sub_body, 0)
            pltpu.sync_copy(hd, h2d_hbm.at[pl.ds(base, 128)])
            return 0

        lax.fori_loop(0, KMAX, chunk_body, 0)

        def u_body(kk, _):
            chunk = w + kk * NW

            @pl.when(chunk < UCH)
            def _():
                base = chunk * 128
                pltpu.sync_copy(idx2_hbm.at[pl.ds(base, 128)], t1)
                pltpu.async_copy(idx1_hbm.at[t1], cdv, semA).wait()
                pltpu.async_copy(batch_hbm.at[cdv], dc, semA).wait()
                pltpu.async_copy(tp_hbm.at[cdv], rp1, semB).wait()
                for j in range(8):
                    s16 = pl.ds(j * 16, 16)
                    rowid = base + j * 16 + lax.iota(jnp.int32, 16)
                    bv[s16] = jnp.where(rowid < N2, dc[s16], jnp.int32(B))
                pltpu.sync_copy(rp1, u2_hbm.at[pl.ds(base, 128)])
                pltpu.sync_copy(bv, b2_hbm.at[pl.ds(base, 128)])
            return 0

        lax.fori_loop(0, 2, u_body, 0)

    return k(X1, TP, idx1, idx2p, batch, src2p, dst2p)


def _mlp1_tc(H1, W1, b1, W2, b2, W3, b3):
    BE = 2000
    G = E1 // BE

    def body(h_ref, w1, bb1, w2, bb2, w3, bb3, out_ref):
        h = h_ref[...]
        h = jnp.maximum(jnp.dot(h, w1[...], preferred_element_type=jnp.float32)
                        + bb1[0:1, :], 0.0)
        h = jnp.maximum(jnp.dot(h, w2[...], preferred_element_type=jnp.float32)
                        + bb2[0:1, :], 0.0)
        out_ref[...] = (jnp.dot(h, w3[...], preferred_element_type=jnp.float32)
                        + bb3[0:1, :])

    return pl.pallas_call(
        body,
        grid=(G,),
        in_specs=[
            pl.BlockSpec((BE, 16), lambda i: (i, 0)),
            pl.BlockSpec((16, 64), lambda i: (0, 0)),
            pl.BlockSpec((8, 64), lambda i: (0, 0)),
            pl.BlockSpec((64, 64), lambda i: (0, 0)),
            pl.BlockSpec((8, 64), lambda i: (0, 0)),
            pl.BlockSpec((64, 128), lambda i: (0, 0)),
            pl.BlockSpec((8, 128), lambda i: (0, 0)),
        ],
        out_specs=pl.BlockSpec((BE, 128), lambda i: (i, 0)),
        out_shape=jax.ShapeDtypeStruct((E1, 128), jnp.float32),
    )(H1, W1, b1, W2, b2, W3, b3)


def _mlp2_tc(H2x, H2d, Wa, Wb, b1, W2, b2, W3, b3):
    BE = 4096
    G = E2P // BE

    def body(hx_ref, hd_ref, wa, wb, bb1, w2, bb2, w3, bb3, out_ref):
        h = jnp.dot(hx_ref[...], wa[...], preferred_element_type=jnp.float32)
        h = h + jnp.dot(hd_ref[...], wb[...], preferred_element_type=jnp.float32)
        h = jnp.maximum(h + bb1[0:1, :], 0.0)
        h = jnp.maximum(jnp.dot(h, w2[...], preferred_element_type=jnp.float32)
                        + bb2[0:1, :], 0.0)
        out_ref[...] = (jnp.dot(h, w3[...], preferred_element_type=jnp.float32)
                        + bb3[0:1, :])

    return pl.pallas_call(
        body,
        grid=(G,),
        in_specs=[
            pl.BlockSpec((BE, 128), lambda i: (i, 0)),
            pl.BlockSpec((BE, 16), lambda i: (i, 0)),
            pl.BlockSpec((128, 128), lambda i: (0, 0)),
            pl.BlockSpec((16, 128), lambda i: (0, 0)),
            pl.BlockSpec((8, 128), lambda i: (0, 0)),
            pl.BlockSpec((128, 128), lambda i: (0, 0)),
            pl.BlockSpec((8, 128), lambda i: (0, 0)),
            pl.BlockSpec((128, 256), lambda i: (0, 0)),
            pl.BlockSpec((8, 256), lambda i: (0, 0)),
        ],
        out_specs=pl.BlockSpec((BE, 256), lambda i: (i, 0)),
        out_shape=jax.ShapeDtypeStruct((E2P, 256), jnp.float32),
    )(H2x, H2d, Wa, Wb, b1, W2, b2, W3, b3)


def _final_tc(X2, U2, B2r, Wa, Wb, b1, W2, b2, W3, b3,
              Wh1, bh1, Wh2, bh2, Wh3, bh3, Wp1, bp1, Wp2, bp2):
    BR = 800
    G = N2P // BR

    def body(x2_ref, u2_ref, b2_ref, wa, wb, bb1, w2, bb2, w3, bb3,
             wh1, bbh1, wh2, bbh2, wh3, bbh3, wp1, bbp1, wp2, bbp2,
             out_ref, acc):
        step = pl.program_id(0)

        @pl.when(step == 0)
        def _():
            acc[...] = jnp.full((B, 1024), _NEG_INF, jnp.float32)

        h = jnp.dot(x2_ref[...], wa[...], preferred_element_type=jnp.float32)
        h = h + jnp.dot(u2_ref[...], wb[...], preferred_element_type=jnp.float32)
        h = jnp.maximum(h + bb1[0:1, :], 0.0)
        h = jnp.maximum(jnp.dot(h, w2[...], preferred_element_type=jnp.float32)
                        + bb2[0:1, :], 0.0)
        h = jnp.dot(h, w3[...], preferred_element_type=jnp.float32) + bb3[0:1, :]
        bid = b2_ref[...][:, 0:1]
        parts = []
        for bb in range(B):
            hm = jnp.where(bid == bb, h, _NEG_INF)
            parts.append(jnp.max(hm, axis=0, keepdims=True))
        acc[...] = jnp.maximum(acc[...], jnp.concatenate(parts, axis=0))

        @pl.when(step == G - 1)
        def _():
            xg = acc[...]
            xg = jnp.where(xg > _NEG_INF, xg, 0.0)
            f = jnp.maximum(jnp.dot(xg, wh1[...], preferred_element_type=jnp.float32)
                            + bbh1[0:1, :], 0.0)
            f = jnp.maximum(jnp.dot(f, wh2[...], preferred_element_type=jnp.float32)
                            + bbh2[0:1, :], 0.0)
            f = jnp.dot(f, wh3[...], preferred_element_type=jnp.float32) + bbh3[0:1, :]
            f = jnp.maximum(jnp.dot(f, wp1[...], preferred_element_type=jnp.float32)
                            + bbp1[0:1, :], 0.0)
            out_ref[...] = (jnp.dot(f, wp2[...], preferred_element_type=jnp.float32)
                            + bbp2[0:1, :])

    full = lambda s: pl.BlockSpec(s, lambda i: (0, 0))
    return pl.pallas_call(
        body,
        grid=(G,),
        in_specs=[
            pl.BlockSpec((BR, 256), lambda i: (i, 0)),
            pl.BlockSpec((BR, 16), lambda i: (i, 0)),
            pl.BlockSpec((BR, 128), lambda i: (i, 0)),
            full((256, 256)), full((16, 256)), full((8, 256)),
            full((256, 512)), full((8, 512)),
            full((512, 1024)), full((8, 1024)),
            full((1024, 512)), full((8, 512)),
            full((512, 256)), full((8, 256)),
            full((256, 32)), full((8, 32)),
            full((32, 32)), full((8, 32)),
            full((32, 8)), full((8, 8)),
        ],
        out_specs=pl.BlockSpec((B, 8), lambda i: (0, 0)),
        out_shape=jax.ShapeDtypeStruct((B, 8), jnp.float32),
        scratch_shapes=[pltpu.VMEM((B, 1024), jnp.float32)],
    )(X2, U2, B2r, Wa, Wb, b1, W2, b2, W3, b3,
      Wh1, bh1, Wh2, bh2, Wh3, bh3, Wp1, bp1, Wp2, bp2)


def kernel(x, pos, batch, idx1, src1, dst1, idx2, src2, dst2,
           params1, params2, params3, params_head, params_policy):
    f32 = jnp.float32
    (W11, b11), (W12, b12), (W13, b13) = params1
    (W21, b21), (W22, b22), (W23, b23) = params2
    (W31, b31), (W32, b32), (W33, b33) = params3
    (Wh1, bh1), (Wh2, bh2), (Wh3, bh3) = params_head
    (Wp1, bp1), (Wp2, bp2) = params_policy

    T = jnp.concatenate([x, pos, jnp.zeros((N, 1), f32)], axis=1)
    TP = jnp.concatenate([jnp.zeros((N, 12), f32), pos,
                          jnp.zeros((N, 1), f32)], axis=1)

    H1 = _gather_h1(T, TP, idx1, src1, dst1)
    W1p = jnp.concatenate([W11, jnp.zeros((1, 64), f32)], axis=0)
    Y1 = _mlp1_tc(H1, W1p, _tile8(b11), W12, _tile8(b12), W13, _tile8(b13))
    dst1p = jnp.concatenate([dst1, jnp.full((409600 - E1,), PAD_DST, jnp.int32)])
    X1 = _scatter_max(Y1, dst1p, N1P, 128, 200)

    idx2p = jnp.concatenate([idx2, jnp.zeros((N2P - N2,), jnp.int32)])
    src2p = jnp.concatenate([src2, jnp.zeros((E2P - E2,), jnp.int32)])
    dst2p = jnp.concatenate([dst2, jnp.full((E2P - E2,), PAD_DST, jnp.int32)])
    H2x, H2d, U2, B2 = _gather_l2(X1, TP, idx1, idx2p, batch, src2p, dst2p)

    W2a = W21[:128, :]
    W2b = jnp.zeros((16, 128), f32).at[12:15, :].set(W21[128:131, :])
    Y2 = _mlp2_tc(H2x, H2d, W2a, W2b, _tile8(b21), W22, _tile8(b22),
                  W23, _tile8(b23))
    X2 = _scatter_max(Y2, dst2p, N2P, 256, 98)

    W3a = W31[:256, :]
    W3b = jnp.zeros((16, 256), f32).at[12:15, :].set(W31[256:259, :])
    B2r = jnp.broadcast_to(B2[:, None], (N2P, 128))
    logits = _final_tc(X2, U2, B2r, W3a, W3b, _tile8(b31), W32, _tile8(b32),
                       W33, _tile8(b33), Wh1, _tile8(bh1), Wh2, _tile8(bh2),
                       Wh3, _tile8(bh3), Wp1, _tile8(bp1), Wp2, _tile8(bp2))
    return logits


# speedup vs baseline: 7.3303x; 1.2873x over previous
import functools

import jax
import jax.numpy as jnp
from jax import lax
from jax.experimental import pallas as pl
from jax.experimental.pallas import tpu as pltpu
from jax.experimental.pallas import tpu_sc as plsc

N = 50000
N1 = 25000
N2 = 6250
E1 = 400000
E2 = 200000
B = 16

NC = 2
NS = 16
NW = NC * NS

N1P = 25088
N2P = 6400
E2P = 200704
PAD_DST = 1 << 20

_NEG_INF = float("-inf")


def _mesh():
    return plsc.VectorSubcoreMesh(core_axis_name="c", subcore_axis_name="s")


def _wid():
    return lax.axis_index("s") * NC + lax.axis_index("c")


def _tile8(b):
    return jnp.broadcast_to(b[None, :], (8, b.shape[0]))


def _gather_h1(T, TP, idx1, src1, dst1):
    CHUNKS = E1 // 128
    KMAX = (CHUNKS + NW - 1) // NW

    @functools.partial(
        pl.kernel,
        out_type=jax.ShapeDtypeStruct((E1, 16), jnp.float32),
        mesh=_mesh(),
        compiler_params=pltpu.CompilerParams(use_tc_tiling_on_sc=False, needs_layout_passes=False),
        scratch_types=[
            pltpu.VMEM((128,), jnp.int32),
            pltpu.VMEM((128,), jnp.int32),
            pltpu.VMEM((128,), jnp.int32),
            pltpu.VMEM((128, 16), jnp.float32),
            pltpu.VMEM((128, 16), jnp.float32),
            pltpu.VMEM((128, 16), jnp.float32),
            pltpu.SemaphoreType.DMA,
            pltpu.SemaphoreType.DMA,
        ],
    )
    def k(T_hbm, TP_hbm, idx1_hbm, src1_hbm, dst1_hbm, h1_hbm,
          sv, dv, ds_v, ra, rb, hb, semA, semB):
        w = _wid()

        def chunk_body(kk, _):
            chunk = w + kk * NW

            @pl.when(chunk < CHUNKS)
            def _():
                base = chunk * 128
                pltpu.sync_copy(src1_hbm.at[pl.ds(base, 128)], sv)
                pltpu.sync_copy(dst1_hbm.at[pl.ds(base, 128)], dv)
                pltpu.async_copy(idx1_hbm.at[dv], ds_v, semA).wait()
                cpA = pltpu.async_copy(T_hbm.at[sv], ra, semA)
                cpB = pltpu.async_copy(TP_hbm.at[ds_v], rb, semB)
                cpA.wait()
                cpB.wait()

                def sub_body(i, c):
                    hb[i, :] = ra[i, :] - rb[i, :]
                    return c
                lax.fori_loop(0, 128, sub_body, 0)
                pltpu.sync_copy(hb, h1_hbm.at[pl.ds(base, 128)])
            return 0

        lax.fori_loop(0, KMAX, chunk_body, 0)

    return k(T, TP, idx1, src1, dst1)


def _scatter_max(Y, dst, nout_p, F, NCHUNK):
    RPW = nout_p // NW
    GB = 16
    CH = 4096

    @functools.partial(
        pl.kernel,
        out_type=jax.ShapeDtypeStruct((nout_p, F), jnp.float32),
        mesh=_mesh(),
        compiler_params=pltpu.CompilerParams(needs_layout_passes=False),
        scratch_types=[
            pltpu.VMEM((RPW + 1, F), jnp.float32),
            pltpu.VMEM((CH,), jnp.int32),
            pltpu.VMEM((CH + GB,), jnp.int32),
            pltpu.VMEM((CH + GB,), jnp.int32),
            pltpu.VMEM((GB, F), jnp.float32),
            pltpu.VMEM((GB, F), jnp.float32),
            pltpu.SemaphoreType.DMA,
            pltpu.SemaphoreType.DMA,
        ],
    )
    def k(y_hbm, dst_hbm, out_hbm, acc, dstbuf, midbuf, lrbuf, rows0, rows1,
          gsem0, gsem1):
        w = _wid()
        lo = w * RPW

        def init_body(r, c):
            for cc in range(F // 16):
                acc[r, pl.ds(cc * 16, 16)] = jnp.full((16,), _NEG_INF, jnp.float32)
            return c
        lax.fori_loop(0, RPW + 1, init_body, 0)

        def acc_batch(bb, rref):
            def jj_body(jj, c):
                lv = lrbuf[pl.ds(bb * GB + jj * 16, 16)]
                for i in range(16):
                    lr = lv[i]
                    for cc in range(F // 16):
                        sl = pl.ds(cc * 16, 16)
                        acc[lr, sl] = jnp.maximum(acc[lr, sl],
                                                  rref[jj * 16 + i, sl])
                return c
            lax.fori_loop(0, GB // 16, jj_body, 0)

        def chunk_body(kk, _):
            pltpu.sync_copy(dst_hbm.at[pl.ds(kk * CH, CH)], dstbuf)

            def scan_body(j, ptr):
                groups = []
                for g in range(8):
                    off = j * 128 + g * 16
                    d = dstbuf[pl.ds(off, 16)]
                    lrel = d - lo
                    m = (lrel >= 0) & (lrel < RPW)
                    eid = kk * CH + off + lax.iota(jnp.int32, 16)
                    key = jnp.where(m, lrel, jnp.int32(1 << 30))
                    sk, sval = plsc.sort_key_val(key, eid)
                    cnt = plsc.all_reduce_population_count(m)
                    groups.append((sk, sval, cnt))
                for sk, sval, cnt in groups:
                    lrbuf[pl.ds(ptr, 16)] = sk
                    midbuf[pl.ds(ptr, 16)] = sval
                    ptr = ptr + cnt[0]
                return ptr

            ptr = lax.fori_loop(0, CH // 128, scan_body, jnp.int32(0))

            zz = jnp.zeros((16,), jnp.int32)
            tt = jnp.full((16,), RPW, jnp.int32)
            for q in range(GB // 16):
                midbuf[pl.ds(ptr + q * 16, 16)] = zz
                lrbuf[pl.ds(ptr + q * 16, 16)] = tt

            nsub = jnp.maximum(lax.div(ptr + (GB - 1), jnp.int32(GB)),
                               jnp.int32(1))

            pltpu.async_copy(
                y_hbm.at[midbuf.at[pl.ds(0, GB)]], rows0, gsem0)

            @pl.when(nsub > 1)
            def _():
                pltpu.async_copy(
                    y_hbm.at[midbuf.at[pl.ds(GB, GB)]], rows1, gsem1)

            pltpu.make_async_copy(y_hbm.at[pl.ds(0, GB)], rows0, gsem0).wait()
            acc_batch(0, rows0)

            @pl.when(nsub > 1)
            def _():
                pltpu.make_async_copy(
                    y_hbm.at[pl.ds(0, GB)], rows1, gsem1).wait()
                acc_batch(1, rows1)

            def tail_body(bb, c):
                pltpu.async_copy(
                    y_hbm.at[midbuf.at[pl.ds(bb * GB, GB)]], rows0, gsem0)
                pltpu.make_async_copy(
                    y_hbm.at[pl.ds(0, GB)], rows0, gsem0).wait()
                acc_batch(bb, rows0)
                return c

            lax.fori_loop(2, nsub, tail_body, 0)
            return 0

        lax.fori_loop(0, NCHUNK, chunk_body, 0)

        def fin_body(r, c):
            for cc in range(F // 16):
                sl = pl.ds(cc * 16, 16)
                v = acc[r, sl]
                acc[r, sl] = jnp.where(v > _NEG_INF, v, jnp.float32(0.0))
            return c
        lax.fori_loop(0, RPW, fin_body, 0)
        pltpu.sync_copy(acc.at[pl.ds(0, RPW)], out_hbm.at[pl.ds(lo, RPW)])

    return k(Y, dst)


def _gather_l2(X1, TP, idx1, idx2p, batch, src2p, dst2p):
    CHUNKS = E2P // 128
    KMAX = CHUNKS // NW
    UCH = N2P // 128

    out_type = [
        jax.ShapeDtypeStruct((E2P, 128), jnp.float32),
        jax.ShapeDtypeStruct((E2P, 16), jnp.float32),
        jax.ShapeDtypeStruct((N2P, 16), jnp.float32),
        jax.ShapeDtypeStruct((N2P,), jnp.int32),
    ]

    @functools.partial(
        pl.kernel,
        out_type=out_type,
        mesh=_mesh(),
        compiler_params=pltpu.CompilerParams(use_tc_tiling_on_sc=False, needs_layout_passes=False),
        scratch_types=[
            pltpu.VMEM((128,), jnp.int32),
            pltpu.VMEM((128,), jnp.int32),
            pltpu.VMEM((128,), jnp.int32),
            pltpu.VMEM((128,), jnp.int32),
            pltpu.VMEM((128,), jnp.int32),
            pltpu.VMEM((128,), jnp.int32),
            pltpu.VMEM((128, 128), jnp.float32),
            pltpu.VMEM((128, 16), jnp.float32),
            pltpu.VMEM((128, 16), jnp.float32),
            pltpu.VMEM((128, 16), jnp.float32),
            pltpu.VMEM((128,), jnp.int32),
            pltpu.SemaphoreType.DMA,
            pltpu.SemaphoreType.DMA,
            pltpu.SemaphoreType.DMA,
        ],
    )
    def k(x1_hbm, tp_hbm, idx1_hbm, idx2_hbm, batch_hbm, src2_hbm, dst2_hbm,
          h2x_hbm, h2d_hbm, u2_hbm, b2_hbm,
          sv, dv, dc, t1, csv, cdv, rx, rp1, rp2, hd, bv,
          semX, semA, semB):
        w = _wid()

        def chunk_body(kk, _):
            base = (w + kk * NW) * 128
            pltpu.sync_copy(src2_hbm.at[pl.ds(base, 128)], sv)
            pltpu.sync_copy(dst2_hbm.at[pl.ds(base, 128)], dv)
            cpX = pltpu.async_copy(x1_hbm.at[sv], rx, semX)
            for j in range(8):
                s16 = pl.ds(j * 16, 16)
                dc[s16] = jnp.minimum(dv[s16], jnp.int32(N2 - 1))
            cpS = pltpu.async_copy(idx1_hbm.at[sv], csv, semA)
            pltpu.async_copy(idx2_hbm.at[dc], t1, semB).wait()
            pltpu.async_copy(idx1_hbm.at[t1], cdv, semB).wait()
            cpS.wait()
            cpA = pltpu.async_copy(tp_hbm.at[csv], rp1, semA)
            cpB = pltpu.async_copy(tp_hbm.at[cdv], rp2, semB)
            cpX.wait()
            pltpu.sync_copy(rx, h2x_hbm.at[pl.ds(base, 128)])
            cpA.wait()
            cpB.wait()

            def sub_body(i, c):
                hd[i, :] = rp1[i, :] - rp2[i, :]
                return c
            lax.fori_loop(0, 128, sub_body, 0)
            pltpu.sync_copy(hd, h2d_hbm.at[pl.ds(base, 128)])
            return 0

        lax.fori_loop(0, KMAX, chunk_body, 0)

        def u_body(kk, _):
            chunk = w + kk * NW

            @pl.when(chunk < UCH)
            def _():
                base = chunk * 128
                pltpu.sync_copy(idx2_hbm.at[pl.ds(base, 128)], t1)
                pltpu.async_copy(idx1_hbm.at[t1], cdv, semA).wait()
                pltpu.async_copy(batch_hbm.at[cdv], dc, semA).wait()
                pltpu.async_copy(tp_hbm.at[cdv], rp1, semB).wait()
                for j in range(8):
                    s16 = pl.ds(j * 16, 16)
                    rowid = base + j * 16 + lax.iota(jnp.int32, 16)
                    bv[s16] = jnp.where(rowid < N2, dc[s16], jnp.int32(B))
                pltpu.sync_copy(rp1, u2_hbm.at[pl.ds(base, 128)])
                pltpu.sync_copy(bv, b2_hbm.at[pl.ds(base, 128)])
            return 0

        lax.fori_loop(0, 2, u_body, 0)

    return k(X1, TP, idx1, idx2p, batch, src2p, dst2p)


def _mlp1_tc(H1, W1, b1, W2, b2, W3, b3):
    BE = 2000
    G = E1 // BE

    def body(h_ref, w1, bb1, w2, bb2, w3, bb3, out_ref):
        h = h_ref[...]
        h = jnp.maximum(jnp.dot(h, w1[...], preferred_element_type=jnp.float32)
                        + bb1[0:1, :], 0.0)
        h = jnp.maximum(jnp.dot(h, w2[...], preferred_element_type=jnp.float32)
                        + bb2[0:1, :], 0.0)
        out_ref[...] = (jnp.dot(h, w3[...], preferred_element_type=jnp.float32)
                        + bb3[0:1, :])

    return pl.pallas_call(
        body,
        grid=(G,),
        in_specs=[
            pl.BlockSpec((BE, 16), lambda i: (i, 0)),
            pl.BlockSpec((16, 64), lambda i: (0, 0)),
            pl.BlockSpec((8, 64), lambda i: (0, 0)),
            pl.BlockSpec((64, 64), lambda i: (0, 0)),
            pl.BlockSpec((8, 64), lambda i: (0, 0)),
            pl.BlockSpec((64, 128), lambda i: (0, 0)),
            pl.BlockSpec((8, 128), lambda i: (0, 0)),
        ],
        out_specs=pl.BlockSpec((BE, 128), lambda i: (i, 0)),
        out_shape=jax.ShapeDtypeStruct((E1, 128), jnp.float32),
    )(H1, W1, b1, W2, b2, W3, b3)


def _mlp2_tc(H2x, H2d, Wa, Wb, b1, W2, b2, W3, b3):
    BE = 4096
    G = E2P // BE

    def body(hx_ref, hd_ref, wa, wb, bb1, w2, bb2, w3, bb3, out_ref):
        h = jnp.dot(hx_ref[...], wa[...], preferred_element_type=jnp.float32)
        h = h + jnp.dot(hd_ref[...], wb[...], preferred_element_type=jnp.float32)
        h = jnp.maximum(h + bb1[0:1, :], 0.0)
        h = jnp.maximum(jnp.dot(h, w2[...], preferred_element_type=jnp.float32)
                        + bb2[0:1, :], 0.0)
        out_ref[...] = (jnp.dot(h, w3[...], preferred_element_type=jnp.float32)
                        + bb3[0:1, :])

    return pl.pallas_call(
        body,
        grid=(G,),
        in_specs=[
            pl.BlockSpec((BE, 128), lambda i: (i, 0)),
            pl.BlockSpec((BE, 16), lambda i: (i, 0)),
            pl.BlockSpec((128, 128), lambda i: (0, 0)),
            pl.BlockSpec((16, 128), lambda i: (0, 0)),
            pl.BlockSpec((8, 128), lambda i: (0, 0)),
            pl.BlockSpec((128, 128), lambda i: (0, 0)),
            pl.BlockSpec((8, 128), lambda i: (0, 0)),
            pl.BlockSpec((128, 256), lambda i: (0, 0)),
            pl.BlockSpec((8, 256), lambda i: (0, 0)),
        ],
        out_specs=pl.BlockSpec((BE, 256), lambda i: (i, 0)),
        out_shape=jax.ShapeDtypeStruct((E2P, 256), jnp.float32),
    )(H2x, H2d, Wa, Wb, b1, W2, b2, W3, b3)


def _final_tc(X2, U2, B2r, Wa, Wb, b1, W2, b2, W3, b3,
              Wh1, bh1, Wh2, bh2, Wh3, bh3, Wp1, bp1, Wp2, bp2):
    BR = 800
    G = N2P // BR

    def body(x2_ref, u2_ref, b2_ref, wa, wb, bb1, w2, bb2, w3, bb3,
             wh1, bbh1, wh2, bbh2, wh3, bbh3, wp1, bbp1, wp2, bbp2,
             out_ref, acc):
        step = pl.program_id(0)

        @pl.when(step == 0)
        def _():
            acc[...] = jnp.full((B, 1024), _NEG_INF, jnp.float32)

        h = jnp.dot(x2_ref[...], wa[...], preferred_element_type=jnp.float32)
        h = h + jnp.dot(u2_ref[...], wb[...], preferred_element_type=jnp.float32)
        h = jnp.maximum(h + bb1[0:1, :], 0.0)
        h = jnp.maximum(jnp.dot(h, w2[...], preferred_element_type=jnp.float32)
                        + bb2[0:1, :], 0.0)
        h = jnp.dot(h, w3[...], preferred_element_type=jnp.float32) + bb3[0:1, :]
        bid = b2_ref[...][:, 0:1]
        parts = []
        for bb in range(B):
            hm = jnp.where(bid == bb, h, _NEG_INF)
            parts.append(jnp.max(hm, axis=0, keepdims=True))
        acc[...] = jnp.maximum(acc[...], jnp.concatenate(parts, axis=0))

        @pl.when(step == G - 1)
        def _():
            xg = acc[...]
            xg = jnp.where(xg > _NEG_INF, xg, 0.0)
            f = jnp.maximum(jnp.dot(xg, wh1[...], preferred_element_type=jnp.float32)
                            + bbh1[0:1, :], 0.0)
            f = jnp.maximum(jnp.dot(f, wh2[...], preferred_element_type=jnp.float32)
                            + bbh2[0:1, :], 0.0)
            f = jnp.dot(f, wh3[...], preferred_element_type=jnp.float32) + bbh3[0:1, :]
            f = jnp.maximum(jnp.dot(f, wp1[...], preferred_element_type=jnp.float32)
                            + bbp1[0:1, :], 0.0)
            out_ref[...] = (jnp.dot(f, wp2[...], preferred_element_type=jnp.float32)
                            + bbp2[0:1, :])

    full = lambda s: pl.BlockSpec(s, lambda i: (0, 0))
    return pl.pallas_call(
        body,
        grid=(G,),
        in_specs=[
            pl.BlockSpec((BR, 256), lambda i: (i, 0)),
            pl.BlockSpec((BR, 16), lambda i: (i, 0)),
            pl.BlockSpec((BR, 128), lambda i: (i, 0)),
            full((256, 256)), full((16, 256)), full((8, 256)),
            full((256, 512)), full((8, 512)),
            full((512, 1024)), full((8, 1024)),
            full((1024, 512)), full((8, 512)),
            full((512, 256)), full((8, 256)),
            full((256, 32)), full((8, 32)),
            full((32, 32)), full((8, 32)),
            full((32, 8)), full((8, 8)),
        ],
        out_specs=pl.BlockSpec((B, 8), lambda i: (0, 0)),
        out_shape=jax.ShapeDtypeStruct((B, 8), jnp.float32),
        scratch_shapes=[pltpu.VMEM((B, 1024), jnp.float32)],
    )(X2, U2, B2r, Wa, Wb, b1, W2, b2, W3, b3,
      Wh1, bh1, Wh2, bh2, Wh3, bh3, Wp1, bp1, Wp2, bp2)


def kernel(x, pos, batch, idx1, src1, dst1, idx2, src2, dst2,
           params1, params2, params3, params_head, params_policy):
    f32 = jnp.float32
    (W11, b11), (W12, b12), (W13, b13) = params1
    (W21, b21), (W22, b22), (W23, b23) = params2
    (W31, b31), (W32, b32), (W33, b33) = params3
    (Wh1, bh1), (Wh2, bh2), (Wh3, bh3) = params_head
    (Wp1, bp1), (Wp2, bp2) = params_policy

    T = jnp.concatenate([x, pos, jnp.zeros((N, 1), f32)], axis=1)
    TP = jnp.concatenate([jnp.zeros((N, 12), f32), pos,
                          jnp.zeros((N, 1), f32)], axis=1)

    H1 = _gather_h1(T, TP, idx1, src1, dst1)
    W1p = jnp.concatenate([W11, jnp.zeros((1, 64), f32)], axis=0)
    Y1 = _mlp1_tc(H1, W1p, _tile8(b11), W12, _tile8(b12), W13, _tile8(b13))
    dst1p = jnp.concatenate([dst1, jnp.full((409600 - E1,), PAD_DST, jnp.int32)])
    X1 = _scatter_max(Y1, dst1p, N1P, 128, 100)

    idx2p = jnp.concatenate([idx2, jnp.zeros((N2P - N2,), jnp.int32)])
    src2p = jnp.concatenate([src2, jnp.zeros((E2P - E2,), jnp.int32)])
    dst2p = jnp.concatenate([dst2, jnp.full((E2P - E2,), PAD_DST, jnp.int32)])
    H2x, H2d, U2, B2 = _gather_l2(X1, TP, idx1, idx2p, batch, src2p, dst2p)

    W2a = W21[:128, :]
    W2b = jnp.zeros((16, 128), f32).at[12:15, :].set(W21[128:131, :])
    Y2 = _mlp2_tc(H2x, H2d, W2a, W2b, _tile8(b21), W22, _tile8(b22),
                  W23, _tile8(b23))
    X2 = _scatter_max(Y2, dst2p, N2P, 256, 49)

    W3a = W31[:256, :]
    W3b = jnp.zeros((16, 256), f32).at[12:15, :].set(W31[256:259, :])
    B2r = jnp.broadcast_to(B2[:, None], (N2P, 128))
    logits = _final_tc(X2, U2, B2r, W3a, W3b, _tile8(b31), W32, _tile8(b32),
                       W33, _tile8(b33), Wh1, _tile8(bh1), Wh2, _tile8(bh2),
                       Wh3, _tile8(bh3), Wp1, _tile8(bp1), Wp2, _tile8(bp2))
    return logits


# speedup vs baseline: 7.6202x; 1.0395x over previous
import functools

import jax
import jax.numpy as jnp
from jax import lax
from jax.experimental import pallas as pl
from jax.experimental.pallas import tpu as pltpu
from jax.experimental.pallas import tpu_sc as plsc

N = 50000
N1 = 25000
N2 = 6250
E1 = 400000
E2 = 200000
B = 16

NC = 2
NS = 16
NW = NC * NS

N1P = 25088
N2P = 6400
E2P = 200704
PAD_DST = 1 << 20

_NEG_INF = float("-inf")


def _mesh():
    return plsc.VectorSubcoreMesh(core_axis_name="c", subcore_axis_name="s")


def _wid():
    return lax.axis_index("s") * NC + lax.axis_index("c")


def _tile8(b):
    return jnp.broadcast_to(b[None, :], (8, b.shape[0]))


def _gather_h1(T, TP, idx1, src1, dst1):
    CHUNKS = E1 // 128
    KMAX = (CHUNKS + NW - 1) // NW

    @functools.partial(
        pl.kernel,
        out_type=jax.ShapeDtypeStruct((E1, 16), jnp.float32),
        mesh=_mesh(),
        compiler_params=pltpu.CompilerParams(use_tc_tiling_on_sc=False, needs_layout_passes=False),
        scratch_types=[
            pltpu.VMEM((128,), jnp.int32),
            pltpu.VMEM((128,), jnp.int32),
            pltpu.VMEM((128,), jnp.int32),
            pltpu.VMEM((128, 16), jnp.float32),
            pltpu.VMEM((128, 16), jnp.float32),
            pltpu.VMEM((128, 16), jnp.float32),
            pltpu.SemaphoreType.DMA,
            pltpu.SemaphoreType.DMA,
        ],
    )
    def k(T_hbm, TP_hbm, idx1_hbm, src1_hbm, dst1_hbm, h1_hbm,
          sv, dv, ds_v, ra, rb, hb, semA, semB):
        w = _wid()

        def chunk_body(kk, _):
            chunk = w + kk * NW

            @pl.when(chunk < CHUNKS)
            def _():
                base = chunk * 128
                pltpu.sync_copy(src1_hbm.at[pl.ds(base, 128)], sv)
                pltpu.sync_copy(dst1_hbm.at[pl.ds(base, 128)], dv)
                pltpu.async_copy(idx1_hbm.at[dv], ds_v, semA).wait()
                cpA = pltpu.async_copy(T_hbm.at[sv], ra, semA)
                cpB = pltpu.async_copy(TP_hbm.at[ds_v], rb, semB)
                cpA.wait()
                cpB.wait()

                def sub_body(i, c):
                    hb[i, :] = ra[i, :] - rb[i, :]
                    return c
                lax.fori_loop(0, 128, sub_body, 0)
                pltpu.sync_copy(hb, h1_hbm.at[pl.ds(base, 128)])
            return 0

        lax.fori_loop(0, KMAX, chunk_body, 0)

    return k(T, TP, idx1, src1, dst1)


def _scatter_max(Y, dst, nout_p, F, NCHUNK, CH):
    RPW = nout_p // NW
    GB = 16

    @functools.partial(
        pl.kernel,
        out_type=jax.ShapeDtypeStruct((nout_p, F), jnp.float32),
        mesh=_mesh(),
        compiler_params=pltpu.CompilerParams(needs_layout_passes=False),
        scratch_types=[
            pltpu.VMEM((RPW + 1, F), jnp.float32),
            pltpu.VMEM((CH,), jnp.int32),
            pltpu.VMEM((CH + GB,), jnp.int32),
            pltpu.VMEM((CH + GB,), jnp.int32),
            pltpu.VMEM((GB, F), jnp.float32),
            pltpu.VMEM((GB, F), jnp.float32),
            pltpu.SemaphoreType.DMA,
            pltpu.SemaphoreType.DMA,
        ],
    )
    def k(y_hbm, dst_hbm, out_hbm, acc, dstbuf, midbuf, lrbuf, rows0, rows1,
          gsem0, gsem1):
        w = _wid()
        lo = w * RPW

        def init_body(r, c):
            for cc in range(F // 16):
                acc[r, pl.ds(cc * 16, 16)] = jnp.full((16,), _NEG_INF, jnp.float32)
            return c
        lax.fori_loop(0, RPW + 1, init_body, 0)

        def acc_batch(bb, rref):
            def jj_body(jj, c):
                lv = lrbuf[pl.ds(bb * GB + jj * 16, 16)]
                for i in range(16):
                    lr = lv[i]
                    for cc in range(F // 16):
                        sl = pl.ds(cc * 16, 16)
                        acc[lr, sl] = jnp.maximum(acc[lr, sl],
                                                  rref[jj * 16 + i, sl])
                return c
            lax.fori_loop(0, GB // 16, jj_body, 0)

        def chunk_body(kk, _):
            pltpu.sync_copy(dst_hbm.at[pl.ds(kk * CH, CH)], dstbuf)

            def scan_body(j, ptr):
                groups = []
                for g in range(8):
                    off = j * 128 + g * 16
                    d = dstbuf[pl.ds(off, 16)]
                    lrel = d - lo
                    m = (lrel >= 0) & (lrel < RPW)
                    eid = kk * CH + off + lax.iota(jnp.int32, 16)
                    key = jnp.where(m, lrel, jnp.int32(1 << 30))
                    sk, sval = plsc.sort_key_val(key, eid)
                    cnt = plsc.all_reduce_population_count(m)
                    groups.append((sk, sval, cnt))
                for sk, sval, cnt in groups:
                    lrbuf[pl.ds(ptr, 16)] = sk
                    midbuf[pl.ds(ptr, 16)] = sval
                    ptr = ptr + cnt[0]
                return ptr

            ptr = lax.fori_loop(0, CH // 128, scan_body, jnp.int32(0))

            zz = jnp.zeros((16,), jnp.int32)
            tt = jnp.full((16,), RPW, jnp.int32)
            for q in range(GB // 16):
                midbuf[pl.ds(ptr + q * 16, 16)] = zz
                lrbuf[pl.ds(ptr + q * 16, 16)] = tt

            nsub = jnp.maximum(lax.div(ptr + (GB - 1), jnp.int32(GB)),
                               jnp.int32(1))

            pltpu.async_copy(
                y_hbm.at[midbuf.at[pl.ds(0, GB)]], rows0, gsem0)

            @pl.when(nsub > 1)
            def _():
                pltpu.async_copy(
                    y_hbm.at[midbuf.at[pl.ds(GB, GB)]], rows1, gsem1)

            pltpu.make_async_copy(y_hbm.at[pl.ds(0, GB)], rows0, gsem0).wait()
            acc_batch(0, rows0)

            @pl.when(nsub > 1)
            def _():
                pltpu.make_async_copy(
                    y_hbm.at[pl.ds(0, GB)], rows1, gsem1).wait()
                acc_batch(1, rows1)

            def tail_body(bb, c):
                pltpu.async_copy(
                    y_hbm.at[midbuf.at[pl.ds(bb * GB, GB)]], rows0, gsem0)
                pltpu.make_async_copy(
                    y_hbm.at[pl.ds(0, GB)], rows0, gsem0).wait()
                acc_batch(bb, rows0)
                return c

            lax.fori_loop(2, nsub, tail_body, 0)
            return 0

        lax.fori_loop(0, NCHUNK, chunk_body, 0)

        def fin_body(r, c):
            for cc in range(F // 16):
                sl = pl.ds(cc * 16, 16)
                v = acc[r, sl]
                acc[r, sl] = jnp.where(v > _NEG_INF, v, jnp.float32(0.0))
            return c
        lax.fori_loop(0, RPW, fin_body, 0)
        pltpu.sync_copy(acc.at[pl.ds(0, RPW)], out_hbm.at[pl.ds(lo, RPW)])

    return k(Y, dst)


def _gather_l2(X1, TP, idx1, idx2p, batch, src2p, dst2p):
    CHUNKS = E2P // 128
    KMAX = CHUNKS // NW
    UCH = N2P // 128

    out_type = [
        jax.ShapeDtypeStruct((E2P, 128), jnp.float32),
        jax.ShapeDtypeStruct((E2P, 16), jnp.float32),
        jax.ShapeDtypeStruct((N2P, 16), jnp.float32),
        jax.ShapeDtypeStruct((N2P,), jnp.int32),
    ]

    @functools.partial(
        pl.kernel,
        out_type=out_type,
        mesh=_mesh(),
        compiler_params=pltpu.CompilerParams(use_tc_tiling_on_sc=False, needs_layout_passes=False),
        scratch_types=[
            pltpu.VMEM((128,), jnp.int32),
            pltpu.VMEM((128,), jnp.int32),
            pltpu.VMEM((128,), jnp.int32),
            pltpu.VMEM((128,), jnp.int32),
            pltpu.VMEM((128,), jnp.int32),
            pltpu.VMEM((128,), jnp.int32),
            pltpu.VMEM((128, 128), jnp.float32),
            pltpu.VMEM((128, 16), jnp.float32),
            pltpu.VMEM((128, 16), jnp.float32),
            pltpu.VMEM((128, 16), jnp.float32),
            pltpu.VMEM((128,), jnp.int32),
            pltpu.SemaphoreType.DMA,
            pltpu.SemaphoreType.DMA,
            pltpu.SemaphoreType.DMA,
        ],
    )
    def k(x1_hbm, tp_hbm, idx1_hbm, idx2_hbm, batch_hbm, src2_hbm, dst2_hbm,
          h2x_hbm, h2d_hbm, u2_hbm, b2_hbm,
          sv, dv, dc, t1, csv, cdv, rx, rp1, rp2, hd, bv,
          semX, semA, semB):
        w = _wid()

        def chunk_body(kk, _):
            base = (w + kk * NW) * 128
            pltpu.sync_copy(src2_hbm.at[pl.ds(base, 128)], sv)
            pltpu.sync_copy(dst2_hbm.at[pl.ds(base, 128)], dv)
            cpX = pltpu.async_copy(x1_hbm.at[sv], rx, semX)
            for j in range(8):
                s16 = pl.ds(j * 16, 16)
                dc[s16] = jnp.minimum(dv[s16], jnp.int32(N2 - 1))
            cpS = pltpu.async_copy(idx1_hbm.at[sv], csv, semA)
            pltpu.async_copy(idx2_hbm.at[dc], t1, semB).wait()
            pltpu.async_copy(idx1_hbm.at[t1], cdv, semB).wait()
            cpS.wait()
            cpA = pltpu.async_copy(tp_hbm.at[csv], rp1, semA)
            cpB = pltpu.async_copy(tp_hbm.at[cdv], rp2, semB)
            cpX.wait()
            pltpu.sync_copy(rx, h2x_hbm.at[pl.ds(base, 128)])
            cpA.wait()
            cpB.wait()

            def sub_body(i, c):
                hd[i, :] = rp1[i, :] - rp2[i, :]
                return c
            lax.fori_loop(0, 128, sub_body, 0)
            pltpu.sync_copy(hd, h2d_hbm.at[pl.ds(base, 128)])
            return 0

        lax.fori_loop(0, KMAX, chunk_body, 0)

        def u_body(kk, _):
            chunk = w + kk * NW

            @pl.when(chunk < UCH)
            def _():
                base = chunk * 128
                pltpu.sync_copy(idx2_hbm.at[pl.ds(base, 128)], t1)
                pltpu.async_copy(idx1_hbm.at[t1], cdv, semA).wait()
                pltpu.async_copy(batch_hbm.at[cdv], dc, semA).wait()
                pltpu.async_copy(tp_hbm.at[cdv], rp1, semB).wait()
                for j in range(8):
                    s16 = pl.ds(j * 16, 16)
                    rowid = base + j * 16 + lax.iota(jnp.int32, 16)
                    bv[s16] = jnp.where(rowid < N2, dc[s16], jnp.int32(B))
                pltpu.sync_copy(rp1, u2_hbm.at[pl.ds(base, 128)])
                pltpu.sync_copy(bv, b2_hbm.at[pl.ds(base, 128)])
            return 0

        lax.fori_loop(0, 2, u_body, 0)

    return k(X1, TP, idx1, idx2p, batch, src2p, dst2p)


def _mlp1_tc(H1, W1, b1, W2, b2, W3, b3):
    BE = 2000
    G = E1 // BE

    def body(h_ref, w1, bb1, w2, bb2, w3, bb3, out_ref):
        h = h_ref[...]
        h = jnp.maximum(jnp.dot(h, w1[...], preferred_element_type=jnp.float32)
                        + bb1[0:1, :], 0.0)
        h = jnp.maximum(jnp.dot(h, w2[...], preferred_element_type=jnp.float32)
                        + bb2[0:1, :], 0.0)
        out_ref[...] = (jnp.dot(h, w3[...], preferred_element_type=jnp.float32)
                        + bb3[0:1, :])

    return pl.pallas_call(
        body,
        grid=(G,),
        in_specs=[
            pl.BlockSpec((BE, 16), lambda i: (i, 0)),
            pl.BlockSpec((16, 64), lambda i: (0, 0)),
            pl.BlockSpec((8, 64), lambda i: (0, 0)),
            pl.BlockSpec((64, 64), lambda i: (0, 0)),
            pl.BlockSpec((8, 64), lambda i: (0, 0)),
            pl.BlockSpec((64, 128), lambda i: (0, 0)),
            pl.BlockSpec((8, 128), lambda i: (0, 0)),
        ],
        out_specs=pl.BlockSpec((BE, 128), lambda i: (i, 0)),
        out_shape=jax.ShapeDtypeStruct((E1, 128), jnp.float32),
    )(H1, W1, b1, W2, b2, W3, b3)


def _mlp2_tc(H2x, H2d, Wa, Wb, b1, W2, b2, W3, b3):
    BE = 4096
    G = E2P // BE

    def body(hx_ref, hd_ref, wa, wb, bb1, w2, bb2, w3, bb3, out_ref):
        h = jnp.dot(hx_ref[...], wa[...], preferred_element_type=jnp.float32)
        h = h + jnp.dot(hd_ref[...], wb[...], preferred_element_type=jnp.float32)
        h = jnp.maximum(h + bb1[0:1, :], 0.0)
        h = jnp.maximum(jnp.dot(h, w2[...], preferred_element_type=jnp.float32)
                        + bb2[0:1, :], 0.0)
        out_ref[...] = (jnp.dot(h, w3[...], preferred_element_type=jnp.float32)
                        + bb3[0:1, :])

    return pl.pallas_call(
        body,
        grid=(G,),
        in_specs=[
            pl.BlockSpec((BE, 128), lambda i: (i, 0)),
            pl.BlockSpec((BE, 16), lambda i: (i, 0)),
            pl.BlockSpec((128, 128), lambda i: (0, 0)),
            pl.BlockSpec((16, 128), lambda i: (0, 0)),
            pl.BlockSpec((8, 128), lambda i: (0, 0)),
            pl.BlockSpec((128, 128), lambda i: (0, 0)),
            pl.BlockSpec((8, 128), lambda i: (0, 0)),
            pl.BlockSpec((128, 256), lambda i: (0, 0)),
            pl.BlockSpec((8, 256), lambda i: (0, 0)),
        ],
        out_specs=pl.BlockSpec((BE, 256), lambda i: (i, 0)),
        out_shape=jax.ShapeDtypeStruct((E2P, 256), jnp.float32),
    )(H2x, H2d, Wa, Wb, b1, W2, b2, W3, b3)


def _final_tc(X2, U2, B2r, Wa, Wb, b1, W2, b2, W3, b3,
              Wh1, bh1, Wh2, bh2, Wh3, bh3, Wp1, bp1, Wp2, bp2):
    BR = 800
    G = N2P // BR

    def body(x2_ref, u2_ref, b2_ref, wa, wb, bb1, w2, bb2, w3, bb3,
             wh1, bbh1, wh2, bbh2, wh3, bbh3, wp1, bbp1, wp2, bbp2,
             out_ref, acc):
        step = pl.program_id(0)

        @pl.when(step == 0)
        def _():
            acc[...] = jnp.full((B, 1024), _NEG_INF, jnp.float32)

        h = jnp.dot(x2_ref[...], wa[...], preferred_element_type=jnp.float32)
        h = h + jnp.dot(u2_ref[...], wb[...], preferred_element_type=jnp.float32)
        h = jnp.maximum(h + bb1[0:1, :], 0.0)
        h = jnp.maximum(jnp.dot(h, w2[...], preferred_element_type=jnp.float32)
                        + bb2[0:1, :], 0.0)
        h = jnp.dot(h, w3[...], preferred_element_type=jnp.float32) + bb3[0:1, :]
        bid = b2_ref[...][:, 0:1]
        parts = []
        for bb in range(B):
            hm = jnp.where(bid == bb, h, _NEG_INF)
            parts.append(jnp.max(hm, axis=0, keepdims=True))
        acc[...] = jnp.maximum(acc[...], jnp.concatenate(parts, axis=0))

        @pl.when(step == G - 1)
        def _():
            xg = acc[...]
            xg = jnp.where(xg > _NEG_INF, xg, 0.0)
            f = jnp.maximum(jnp.dot(xg, wh1[...], preferred_element_type=jnp.float32)
                            + bbh1[0:1, :], 0.0)
            f = jnp.maximum(jnp.dot(f, wh2[...], preferred_element_type=jnp.float32)
                            + bbh2[0:1, :], 0.0)
            f = jnp.dot(f, wh3[...], preferred_element_type=jnp.float32) + bbh3[0:1, :]
            f = jnp.maximum(jnp.dot(f, wp1[...], preferred_element_type=jnp.float32)
                            + bbp1[0:1, :], 0.0)
            out_ref[...] = (jnp.dot(f, wp2[...], preferred_element_type=jnp.float32)
                            + bbp2[0:1, :])

    full = lambda s: pl.BlockSpec(s, lambda i: (0, 0))
    return pl.pallas_call(
        body,
        grid=(G,),
        in_specs=[
            pl.BlockSpec((BR, 256), lambda i: (i, 0)),
            pl.BlockSpec((BR, 16), lambda i: (i, 0)),
            pl.BlockSpec((BR, 128), lambda i: (i, 0)),
            full((256, 256)), full((16, 256)), full((8, 256)),
            full((256, 512)), full((8, 512)),
            full((512, 1024)), full((8, 1024)),
            full((1024, 512)), full((8, 512)),
            full((512, 256)), full((8, 256)),
            full((256, 32)), full((8, 32)),
            full((32, 32)), full((8, 32)),
            full((32, 8)), full((8, 8)),
        ],
        out_specs=pl.BlockSpec((B, 8), lambda i: (0, 0)),
        out_shape=jax.ShapeDtypeStruct((B, 8), jnp.float32),
        scratch_shapes=[pltpu.VMEM((B, 1024), jnp.float32)],
    )(X2, U2, B2r, Wa, Wb, b1, W2, b2, W3, b3,
      Wh1, bh1, Wh2, bh2, Wh3, bh3, Wp1, bp1, Wp2, bp2)


def kernel(x, pos, batch, idx1, src1, dst1, idx2, src2, dst2,
           params1, params2, params3, params_head, params_policy):
    f32 = jnp.float32
    (W11, b11), (W12, b12), (W13, b13) = params1
    (W21, b21), (W22, b22), (W23, b23) = params2
    (W31, b31), (W32, b32), (W33, b33) = params3
    (Wh1, bh1), (Wh2, bh2), (Wh3, bh3) = params_head
    (Wp1, bp1), (Wp2, bp2) = params_policy

    T = jnp.concatenate([x, pos, jnp.zeros((N, 1), f32)], axis=1)
    TP = jnp.concatenate([jnp.zeros((N, 12), f32), pos,
                          jnp.zeros((N, 1), f32)], axis=1)

    H1 = _gather_h1(T, TP, idx1, src1, dst1)
    W1p = jnp.concatenate([W11, jnp.zeros((1, 64), f32)], axis=0)
    Y1 = _mlp1_tc(H1, W1p, _tile8(b11), W12, _tile8(b12), W13, _tile8(b13))
    dst1p = jnp.concatenate([dst1, jnp.full((409600 - E1,), PAD_DST, jnp.int32)])
    X1 = _scatter_max(Y1, dst1p, N1P, 128, 50, 8192)

    idx2p = jnp.concatenate([idx2, jnp.zeros((N2P - N2,), jnp.int32)])
    src2p = jnp.concatenate([src2, jnp.zeros((E2P - E2,), jnp.int32)])
    dst2p = jnp.concatenate([dst2, jnp.full((E2P - E2,), PAD_DST, jnp.int32)])
    H2x, H2d, U2, B2 = _gather_l2(X1, TP, idx1, idx2p, batch, src2p, dst2p)

    W2a = W21[:128, :]
    W2b = jnp.zeros((16, 128), f32).at[12:15, :].set(W21[128:131, :])
    Y2 = _mlp2_tc(H2x, H2d, W2a, W2b, _tile8(b21), W22, _tile8(b22),
                  W23, _tile8(b23))
    X2 = _scatter_max(Y2, dst2p, N2P, 256, 49, 4096)

    W3a = W31[:256, :]
    W3b = jnp.zeros((16, 256), f32).at[12:15, :].set(W31[256:259, :])
    B2r = jnp.broadcast_to(B2[:, None], (N2P, 128))
    logits = _final_tc(X2, U2, B2r, W3a, W3b, _tile8(b31), W32, _tile8(b32),
                       W33, _tile8(b33), Wh1, _tile8(bh1), Wh2, _tile8(bh2),
                       Wh3, _tile8(bh3), Wp1, _tile8(bp1), Wp2, _tile8(bp2))
    return logits
